# Initial kernel scaffold; baseline (speedup 1.0000x reference)
import jax
import jax.numpy as jnp
from jax.experimental import pallas as pl


def kernel(node_features, rigids_rot, rigids_trans, edge_features, edge_index, seq_edge_features, seq_edge_index, x_mask, noising_mask, node_vectors, params):
    raise NotImplementedError("write your pallas kernel here")


# speedup vs baseline: 13.6802x; 13.6802x over previous
import functools

import jax
import jax.numpy as jnp
from jax import lax
from jax.experimental import pallas as pl
from jax.experimental.pallas import tpu as pltpu
from jax.experimental.pallas import tpu_sc as plsc

N = 10000
E = 160000
CS = 128
CV = 8
CZ = 32
CH = 16
H = 8
PQ = 4
PV = 8

NT = 10240
EP = 163840
NEG = -3e38
HP = lax.Precision.HIGHEST

NWORK = 32
PER_W = EP // NWORK
GCH = 128
SCH = 64
CCOL = 128
NPASS = 5
ROW_W = 640

@functools.cache
def _mesh():
    return plsc.VectorSubcoreMesh(core_axis_name="c", subcore_axis_name="s")


def _wid():
    return lax.axis_index("s") * 2 + lax.axis_index("c")


def _sc_gather(table, idx2d, D, rows):
    per_w = rows // NWORK
    chunks = per_w // GCH

    @functools.partial(
        pl.kernel,
        out_type=jax.ShapeDtypeStruct((rows, D), jnp.float32),
        mesh=_mesh(),
        scratch_types=[
            pltpu.VMEM((chunks, GCH), jnp.int32),
            pltpu.VMEM((GCH, D), jnp.float32),
            pltpu.SemaphoreType.DMA,
        ],
    )
    def k(table_hbm, idx_hbm, out_hbm, idx_v, rows_v, sem):
        w = _wid()
        row0 = w * per_w
        chunk0 = w * chunks
        pltpu.sync_copy(idx_hbm.at[pl.ds(chunk0, chunks)], idx_v)

        def body(c, _):
            pltpu.async_copy(table_hbm.at[idx_v.at[c]], rows_v, sem).wait()
            pltpu.sync_copy(rows_v, out_hbm.at[pl.ds(row0 + c * GCH, GCH)])
            return 0

        lax.fori_loop(0, chunks, body, 0)

    return k(table, idx2d)


def _sc_segmax(lflat, midx):
    CH_E = 512
    chunks = PER_W // CH_E

    @functools.partial(
        pl.kernel,
        out_type=jax.ShapeDtypeStruct((NWORK, NT * 8), jnp.float32),
        mesh=_mesh(),
        compiler_params=pltpu.CompilerParams(needs_layout_passes=False),
        scratch_types=[
            pltpu.VMEM((NT * 8,), jnp.float32),
            pltpu.VMEM((CH_E * 8,), jnp.float32),
            pltpu.VMEM((CH_E * 16,), jnp.int32),
        ],
    )
    def k(l_hbm, mi_hbm, out_hbm, m_v, l_v, i_v):
        w = _wid()
        base = w * PER_W
        negv = jnp.full((16,), NEG, jnp.float32)
        lanes = lax.iota(jnp.int32, 16)
        lo_mask = lanes < 8

        def init(i, _):
            m_v[pl.ds(i * 16, 16)] = negv
            return 0

        lax.fori_loop(0, NT * 8 // 16, init, 0)

        def chunk(c, _):
            pltpu.sync_copy(l_hbm.at[pl.ds((base + c * CH_E) * 8, CH_E * 8)], l_v)
            pltpu.sync_copy(mi_hbm.at[pl.ds((base + c * CH_E) * 16, CH_E * 16)],
                            i_v)

            def edge(e, _):
                iv = i_v[pl.ds(e * 16, 16)]
                lv = l_v[pl.ds(e * 8, 16)]
                lsel = jnp.where(lo_mask, lv, negv)
                mv = plsc.load_gather(m_v, [iv])
                plsc.store_scatter(m_v, [iv], jnp.maximum(mv, lsel), mask=lo_mask)
                return 0

            lax.fori_loop(0, CH_E, edge, 0)
            return 0

        lax.fori_loop(0, chunks, chunk, 0)
        pltpu.sync_copy(m_v, out_hbm.at[w])

    return k(lflat, midx)


def _sc_scatter(wps, idx2d, zeros_blk):
    chunks = PER_W // SCH

    @functools.partial(
        pl.kernel,
        out_type=jax.ShapeDtypeStruct((2 * NPASS * NT, CCOL), jnp.float32),
        mesh=_mesh(),
        scratch_types=[
            pltpu.VMEM_SHARED((NT, CCOL), jnp.float32),
            pltpu.VMEM((chunks, SCH), jnp.int32),
            pltpu.VMEM((SCH, CCOL), jnp.float32),
            pltpu.SemaphoreType.DMA,
        ],
    )
    def k(w0_h, w1_h, w2_h, w3_h, w4_h, idx_h, z_h, out_h, acc, idx_v, v_buf,
          sem):
        cid = lax.axis_index("c")
        sid = lax.axis_index("s")
        w = sid * 2 + cid
        row0 = w * PER_W
        pltpu.sync_copy(idx_h.at[pl.ds(w * chunks, chunks)], idx_v)
        for p, wp in enumerate((w0_h, w1_h, w2_h, w3_h, w4_h)):
            pltpu.sync_copy(z_h, acc.at[pl.ds(sid * 640, 640)])
            plsc.subcore_barrier()

            def chunk(c, _):
                pltpu.async_copy(wp.at[pl.ds(row0 + c * SCH, SCH)], v_buf, sem).wait()
                pltpu.sync_copy(v_buf, acc.at[idx_v.at[c]], add=True)
                return 0

            lax.fori_loop(0, chunks, chunk, 0)
            plsc.subcore_barrier()
            slot = cid * NPASS + p
            pltpu.sync_copy(acc.at[pl.ds(sid * 640, 640)],
                            out_h.at[pl.ds(slot * NT + sid * 640, 640)])
            plsc.subcore_barrier()

    return k(*wps, idx2d, zeros_blk)


def _dot(a, b):
    return jnp.dot(a, b, precision=HP)


def _t1_tables(s, vx, vy, vz, rot9, trans, p):
    B = 512
    grid = NT // B

    def body(s_r, vx_r, vy_r, vz_r, r9_r, tr_r, wq_r, wk_r, wv_r, wqp_r, wkp_r,
             wvp_r, vq_r, vk_r, vv_r, td_r, tsa_r, tsb_r):
        sb = s_r[...]
        vpl = (vx_r[...], vy_r[...], vz_r[...])
        r9 = r9_r[...]
        tr = tr_r[...]
        q = _dot(sb, wq_r[...])
        kk = _dot(sb, wk_r[...])
        vs = _dot(sb, wv_r[...])
        qp = [_dot(sb, wqp_r[j]) + _dot(vpl[j], vq_r[...]) for j in range(3)]
        kp = [_dot(sb, wkp_r[j]) + _dot(vpl[j], vk_r[...]) for j in range(3)]
        vp = [_dot(sb, wvp_r[j]) + _dot(vpl[j], vv_r[...]) for j in range(3)]

        def glob(pts, i):
            return (r9[:, 3 * i:3 * i + 1] * pts[0]
                    + r9[:, 3 * i + 1:3 * i + 2] * pts[1]
                    + r9[:, 3 * i + 2:3 * i + 3] * pts[2]
                    + tr[:, i:i + 1])

        qpg = [glob(qp, i) for i in range(3)]
        kpg = [glob(kp, i) for i in range(3)]
        vpg = [glob(vp, i) for i in range(3)]
        rid = pl.program_id(0) * B + lax.broadcasted_iota(jnp.int32, (B, 1), 0)
        valid = rid < N
        z32 = jnp.zeros((B, 32), jnp.float32)
        td = jnp.where(valid, jnp.concatenate([q] + qpg + [z32], axis=1), 0.0)
        tsa = jnp.where(valid, jnp.concatenate([kk] + kpg + [z32], axis=1), 0.0)
        tsb = jnp.where(valid,
                        jnp.concatenate([vs] + vpg + [z32, z32], axis=1), 0.0)
        td_r[...] = td
        tsa_r[...] = tsa
        tsb_r[...] = tsb

    full = lambda shape: pl.BlockSpec(shape, lambda i: tuple(0 for _ in shape))
    row = lambda w: pl.BlockSpec((B, w), lambda i: (i, 0))
    return pl.pallas_call(
        body,
        grid=(grid,),
        in_specs=[row(CS), row(CV), row(CV), row(CV), row(9), row(3),
                  full((CS, CS)), full((CS, CS)), full((CS, CS)),
                  full((3, CS, H * PQ)), full((3, CS, H * PQ)),
                  full((3, CS, H * PV)),
                  full((CV, H * PQ)), full((CV, H * PQ)), full((CV, H * PV))],
        out_specs=[row(256), row(256), row(384)],
        out_shape=[jax.ShapeDtypeStruct((NT, 256), jnp.float32),
                   jax.ShapeDtypeStruct((NT, 256), jnp.float32),
                   jax.ShapeDtypeStruct((NT, 384), jnp.float32)],
    )(s, vx, vy, vz, rot9, trans, p["wq"], p["wk"], p["wv"], p["wqp3"],
      p["wkp3"], p["wvp3"], p["vq"], p["vk"], p["vv"])


def _t2_logits(td_g, tsa_g, zp, dstcol, p, selqk, seld2):
    B = 512
    grid = EP // B

    def body(td_r, tsa_r, z_r, d_r, wb_r, g_r, sq_r, sd_r, l_r, mi_r):
        td = td_r[...]
        tsa = tsa_r[...]
        z = z_r[...]
        lq = _dot(td[:, :CS] * tsa[:, :CS], sq_r[...]) * 0.25
        d2 = _dot((td[:, CS:224] - tsa[:, CS:224]) ** 2, sd_r[...])
        logits = lq + _dot(z, wb_r[...]) - 0.5 * g_r[...] * d2
        eid = pl.program_id(0) * B + lax.broadcasted_iota(jnp.int32, (B, H), 0)
        l_r[...] = jnp.where(eid < E, logits, NEG)
        mi_r[...] = d_r[...] * 8 + lax.broadcasted_iota(jnp.int32, (B, 16), 1)

    full = lambda shape: pl.BlockSpec(shape, lambda i: tuple(0 for _ in shape))
    row = lambda w: pl.BlockSpec((B, w), lambda i: (i, 0))
    return pl.pallas_call(
        body,
        grid=(grid,),
        in_specs=[row(256), row(256), row(CZ), row(1),
                  full((CZ, H)), full((1, H)), full((CS, H)), full((96, H))],
        out_specs=[row(H), row(16)],
        out_shape=[jax.ShapeDtypeStruct((EP, H), jnp.float32),
                   jax.ShapeDtypeStruct((EP, 16), jnp.int32)],
    )(td_g, tsa_g, zp, dstcol, p["wb"], p["gsp"], selqk, seld2)


def _t3_mmerge(parts):
    B = 512
    grid = NT // B

    def body(p_r, m_r):
        x = p_r[...]
        m = x[0]
        for i in range(1, NWORK):
            m = jnp.maximum(m, x[i])
        m_r[...] = jnp.concatenate([m, jnp.zeros((B, 120), jnp.float32)],
                                   axis=1)

    return pl.pallas_call(
        body,
        grid=(grid,),
        in_specs=[pl.BlockSpec((NWORK, B, 8), lambda i: (0, i, 0))],
        out_specs=pl.BlockSpec((B, 128), lambda i: (i, 0)),
        out_shape=jax.ShapeDtypeStruct((NT, 128), jnp.float32),
    )(parts)


def _t4_weights(logits, m_g, tsb_g, zp, e16, e8, e32):
    B = 512
    grid = EP // B

    def body(l_r, m_r, tsb_r, z_r, e16_r, e8_r, e32_r, w0_r, w1_r, w2_r, w3_r,
             w4_r):
        w = jnp.exp(l_r[...] - m_r[...][:, :8])
        tsb = tsb_r[...]
        z = z_r[...]
        r16 = _dot(w, e16_r[...])
        r8 = _dot(w, e8_r[...])
        r32 = _dot(w, e32_r[...])
        ztile = jnp.concatenate([z] * H, axis=1)
        cat = jnp.concatenate(
            [r16 * tsb[:, :CS]]
            + [r8 * tsb[:, CS + 64 * j:CS + 64 * (j + 1)] for j in range(3)]
            + [r32 * ztile, w, jnp.zeros((B, 56), jnp.float32)], axis=1)
        for i, o_r in enumerate((w0_r, w1_r, w2_r, w3_r, w4_r)):
            o_r[...] = cat[:, i * CCOL:(i + 1) * CCOL]

    full = lambda shape: pl.BlockSpec(shape, lambda i: tuple(0 for _ in shape))
    row = lambda w: pl.BlockSpec((B, w), lambda i: (i, 0))
    return pl.pallas_call(
        body,
        grid=(grid,),
        in_specs=[row(H), row(128), row(384), row(CZ),
                  full((H, 128)), full((H, 64)), full((H, 256))],
        out_specs=[row(CCOL)] * NPASS,
        out_shape=[jax.ShapeDtypeStruct((EP, CCOL), jnp.float32)] * NPASS,
    )(logits, m_g, tsb_g, zp, e16, e8, e32)


def _t5_finalize(acc, s, vx, vy, vz, rot9, trans, inv, p, lnp, vlng,
                 e16, e8, e32):
    B = 400
    grid = N // B

    def body(a_r, s_r, vx_r, vy_r, vz_r, r9_r, tr_r, inv_r, wo_r, bo_r, wpv_r,
             g_r, b_r, vg_r, e16_r, e8_r, e32_r, so_r, vxo_r, vyo_r, vzo_r):
        a = a_r[...]
        acc2 = a[0] + a[1]
        flat = jnp.concatenate([acc2[j] for j in range(NPASS)], axis=1)
        den = flat[:, 576:584] + 1e-9
        d16 = _dot(den, e16_r[...])
        d8 = _dot(den, e8_r[...])
        d32 = _dot(den, e32_r[...])
        o = flat[:, :CS] / d16
        r9 = r9_r[...]
        tr = tr_r[...]
        opt = [flat[:, CS + 64 * j:CS + 64 * (j + 1)] / d8 for j in range(3)]
        optl = [sum((r9[:, 3 * j + i:3 * j + i + 1]
                     * (opt[j] - tr[:, j:j + 1])) for j in range(3))
                for i in range(3)]
        onorm = jnp.sqrt(optl[0] ** 2 + optl[1] ** 2 + optl[2] ** 2 + 1e-8)
        ozn = flat[:, 320:576] / d32
        feats = jnp.concatenate([o] + optl + [onorm, ozn], axis=1)
        su = _dot(feats, wo_r[...]) + bo_r[...]
        inv_b = inv_r[...]
        sn = s_r[...] + su * inv_b
        mu = jnp.mean(sn, axis=1, keepdims=True)
        var = jnp.mean((sn - mu) ** 2, axis=1, keepdims=True)
        so_r[...] = (sn - mu) / jnp.sqrt(var + 1e-5) * g_r[...] + b_r[...]
        vn = [v_r[...] + _dot(optl[i], wpv_r[...]) * inv_b
              for i, v_r in enumerate((vx_r, vy_r, vz_r))]
        n2 = jnp.mean(vn[0] ** 2 + vn[1] ** 2 + vn[2] ** 2, axis=1,
                      keepdims=True)
        scale = vg_r[...] / jnp.sqrt(n2 + 1e-6)
        vxo_r[...] = vn[0] * scale
        vyo_r[...] = vn[1] * scale
        vzo_r[...] = vn[2] * scale

    full = lambda shape: pl.BlockSpec(shape, lambda i: tuple(0 for _ in shape))
    row = lambda w: pl.BlockSpec((B, w), lambda i: (i, 0))
    return pl.pallas_call(
        body,
        grid=(grid,),
        in_specs=[pl.BlockSpec((2, NPASS, B, CCOL), lambda i: (0, 0, i, 0)),
                  row(CS), row(CV), row(CV), row(CV), row(9), row(3), row(1),
                  full((640, CS)), full((1, CS)), full((64, CV)),
                  full((1, CS)), full((1, CS)), full((1, CV)),
                  full((H, 128)), full((H, 64)), full((H, 256))],
        out_specs=[row(CS), row(CV), row(CV), row(CV)],
        out_shape=[jax.ShapeDtypeStruct((N, CS), jnp.float32)]
        + [jax.ShapeDtypeStruct((N, CV), jnp.float32)] * 3,
    )(acc, s, vx, vy, vz, rot9, trans, inv, p["woP"], p["bo"], p["wpv"],
      lnp["g"].reshape(1, CS), lnp["b"].reshape(1, CS), vlng, e16, e8, e32)


def _t6_post(s, vx, vy, vz, rot9, trans, inv, noise, pr):
    B = 400
    grid = N // B

    def body(s_r, vx_r, vy_r, vz_r, r9_r, tr_r, inv_r, no_r,
             lc_w1, lc_b1, lc_ws, lc_bs, lc_wg, lc_bg, lc_wm,
             lr_w1, lr_b1, lr_ws, lr_bs, lr_wg, lr_bg, lr_wm,
             l3g, l3b, v3g,
             nt_w1, nt_b1, nt_w2, nt_b2, nt_g, nt_b, nt_wg, nt_bg, nt_wm,
             w6_r, b6_r, wv2_r,
             so_r, vxo_r, vyo_r, vzo_r, r9o_r, tro_r):
        s_ = s_r[...]
        v = [vx_r[...], vy_r[...], vz_r[...]]
        inv_b = inv_r[...]
        no_b = no_r[...]

        def gate(s_, v, w1, b1, ws, bs, wg, bg, wm, act):
            nrm = jnp.sqrt(v[0] ** 2 + v[1] ** 2 + v[2] ** 2 + 1e-8)
            h = jnp.maximum(_dot(jnp.concatenate([s_, nrm], axis=1), w1[...])
                            + b1[...], 0.0)
            su = _dot(h, ws[...]) + bs[...]
            g = act(_dot(h, wg[...]) + bg[...])
            vu = [g * _dot(v[i], wm[...]) for i in range(3)]
            return su, vu

        su, vu = gate(s_, v, lc_w1, lc_b1, lc_ws, lc_bs, lc_wg, lc_bg, lc_wm,
                      jax.nn.sigmoid)
        s_ = s_ + su * inv_b
        v = [v[i] + vu[i] * inv_b for i in range(3)]
        su, vu = gate(s_, v, lr_w1, lr_b1, lr_ws, lr_bs, lr_wg, lr_bg, lr_wm,
                      jnp.tanh)
        sn = s_ + su * inv_b

        def ln(x, g, b):
            mu = jnp.mean(x, axis=1, keepdims=True)
            var = jnp.mean((x - mu) ** 2, axis=1, keepdims=True)
            return (x - mu) / jnp.sqrt(var + 1e-5) * g[...] + b[...]

        s_ = ln(sn, l3g, l3b)
        v = [v[i] + vu[i] * inv_b for i in range(3)]
        n2 = jnp.mean(v[0] ** 2 + v[1] ** 2 + v[2] ** 2, axis=1, keepdims=True)
        v = [v[i] * (v3g[...] / jnp.sqrt(n2 + 1e-6)) for i in range(3)]

        h = jnp.maximum(_dot(s_, nt_w1[...]) + nt_b1[...], 0.0)
        s2 = ln(s_ + _dot(h, nt_w2[...]) + nt_b2[...], nt_g, nt_b)
        gg = jax.nn.sigmoid(_dot(s2, nt_wg[...]) + nt_bg[...])
        v = [v[i] + gg * _dot(v[i], nt_wm[...]) for i in range(3)]

        s_f = s2 * inv_b
        v_f = [v[i] * inv_b for i in range(3)]

        sb = s_f * no_b
        vb = [v_f[i] * no_b for i in range(3)]
        a = [_dot(vb[i], wv2_r[...]) for i in range(3)]
        vc = jnp.concatenate([a[0][:, :1], a[1][:, :1], a[2][:, :1],
                              a[0][:, 1:], a[1][:, 1:], a[2][:, 1:]], axis=1)
        upd = (_dot(sb, w6_r[...]) + b6_r[...] + vc) * no_b

        qn = jnp.sqrt(1.0 + upd[:, 0:1] ** 2 + upd[:, 1:2] ** 2
                      + upd[:, 2:3] ** 2)
        qw = 1.0 / qn
        qx = upd[:, 0:1] / qn
        qy = upd[:, 1:2] / qn
        qz = upd[:, 2:3] / qn
        ru = [1 - 2 * (qy * qy + qz * qz), 2 * (qx * qy - qz * qw),
              2 * (qx * qz + qy * qw),
              2 * (qx * qy + qz * qw), 1 - 2 * (qx * qx + qz * qz),
              2 * (qy * qz - qx * qw),
              2 * (qx * qz - qy * qw), 2 * (qy * qz + qx * qw),
              1 - 2 * (qx * qx + qy * qy)]
        r9 = r9_r[...]
        newr = [sum(r9[:, 3 * i + j:3 * i + j + 1] * ru[3 * j + k]
                    for j in range(3)) for i in range(3) for k in range(3)]
        tr = tr_r[...]
        newt = [tr[:, i:i + 1]
                + sum(r9[:, 3 * i + j:3 * i + j + 1] * upd[:, 3 + j:4 + j]
                      for j in range(3)) for i in range(3)]
        so_r[...] = s_f
        vxo_r[...] = v_f[0]
        vyo_r[...] = v_f[1]
        vzo_r[...] = v_f[2]
        r9o_r[...] = jnp.concatenate(newr, axis=1)
        tro_r[...] = jnp.concatenate(newt, axis=1)

    full = lambda shape: pl.BlockSpec(shape, lambda i: tuple(0 for _ in shape))
    row = lambda w: pl.BlockSpec((B, w), lambda i: (i, 0))
    lcu, lru, nt, bb = pr["lcu"], pr["lru"], pr["nt"], pr["bb"]
    return pl.pallas_call(
        body,
        grid=(grid,),
        in_specs=[row(CS), row(CV), row(CV), row(CV), row(9), row(3), row(1),
                  row(1),
                  full((CS + CV, CS)), full((1, CS)), full((CS, CS)),
                  full((1, CS)), full((CS, CV)), full((1, CV)), full((CV, CV)),
                  full((CS + CV, CS)), full((1, CS)), full((CS, CS)),
                  full((1, CS)), full((CS, CV)), full((1, CV)), full((CV, CV)),
                  full((1, CS)), full((1, CS)), full((1, CV)),
                  full((CS, 2 * CS)), full((1, 2 * CS)), full((2 * CS, CS)),
                  full((1, CS)), full((1, CS)), full((1, CS)),
                  full((CS, CV)), full((1, CV)), full((CV, CV)),
                  full((CS, 6)), full((1, 6)), full((CV, 2))],
        out_specs=[row(CS), row(CV), row(CV), row(CV), row(9), row(3)],
        out_shape=[jax.ShapeDtypeStruct((N, CS), jnp.float32),
                   jax.ShapeDtypeStruct((N, CV), jnp.float32),
                   jax.ShapeDtypeStruct((N, CV), jnp.float32),
                   jax.ShapeDtypeStruct((N, CV), jnp.float32),
                   jax.ShapeDtypeStruct((N, 9), jnp.float32),
                   jax.ShapeDtypeStruct((N, 3), jnp.float32)],
    )(s, vx, vy, vz, rot9, trans, inv, noise,
      lcu["w1"], lcu["b1"], lcu["ws"], lcu["bs"], lcu["wg"], lcu["bg"],
      lcu["wm"],
      lru["w1"], lru["b1"], lru["ws"], lru["bs"], lru["wg"], lru["bg"],
      lru["wm"],
      pr["ln_s3g"], pr["ln_s3b"], pr["ln_v3"],
      nt["w1"], nt["b1"], nt["w2"], nt["b2"], nt["g"], nt["b"], nt["wg"],
      nt["bg"], nt["wm"],
      bb["w6"], bb["b6"], bb["wv2"])


def _t7_edge_tr(sg, piece_src, piece_dst, z, p):
    B = 512
    grid = (E + B - 1) // B
    off_s = piece_src * (EP // B)
    off_d = piece_dst * (EP // B)

    def body(ss_r, sd_r, z_r, w1_r, b1_r, w2_r, b2_r, g_r, b_r, zo_r):
        z_ = z_r[...]
        hcat = jnp.concatenate([ss_r[...], sd_r[...], z_], axis=1)
        h = jnp.maximum(_dot(hcat, w1_r[...]) + b1_r[...], 0.0)
        zn = z_ + _dot(h, w2_r[...]) + b2_r[...]
        mu = jnp.mean(zn, axis=1, keepdims=True)
        var = jnp.mean((zn - mu) ** 2, axis=1, keepdims=True)
        zo_r[...] = (zn - mu) / jnp.sqrt(var + 1e-5) * g_r[...] + b_r[...]

    full = lambda shape: pl.BlockSpec(shape, lambda i: tuple(0 for _ in shape))
    return pl.pallas_call(
        body,
        grid=(grid,),
        in_specs=[pl.BlockSpec((B, CS), lambda i: (i + off_s, 0)),
                  pl.BlockSpec((B, CS), lambda i: (i + off_d, 0)),
                  pl.BlockSpec((B, CZ), lambda i: (i, 0)),
                  full((2 * CS + CZ, 2 * CZ)), full((1, 2 * CZ)),
                  full((2 * CZ, CZ)), full((1, CZ)),
                  full((1, CZ)), full((1, CZ))],
        out_specs=pl.BlockSpec((B, CZ), lambda i: (i, 0)),
        out_shape=jax.ShapeDtypeStruct((E, CZ), jnp.float32),
    )(sg, sg, z, p["w1"], p["b1"], p["w2"], p["b2"], p["g"], p["b"])


def _prep_psa(p):
    wo = p["wo"]
    o_part = wo[:CS]
    optl_part = wo[CS:CS + 192].reshape(64, 3, CS).transpose(1, 0, 2).reshape(192, CS)
    rest = wo[CS + 192:]
    return {
        "wq": p["wq"], "wk": p["wk"], "wv": p["wv"],
        "wqp3": p["wqp"].reshape(CS, H * PQ, 3).transpose(2, 0, 1),
        "wkp3": p["wkp"].reshape(CS, H * PQ, 3).transpose(2, 0, 1),
        "wvp3": p["wvp"].reshape(CS, H * PV, 3).transpose(2, 0, 1),
        "vq": p["vq"], "vk": p["vk"], "vv": p["vv"],
        "wb": p["wb"],
        "gsp": jax.nn.softplus(p["gamma"]).reshape(1, H),
        "woP": jnp.concatenate([o_part, optl_part, rest], axis=0),
        "bo": p["bo"].reshape(1, CS),
        "wpv": p["wpv"],
    }


def _pad_idx(idx, pad_val, rows):
    return jnp.pad(idx, (0, rows - idx.shape[0]), constant_values=pad_val)


def _attention(s, vx, vy, vz, rot9, trans, inv, z, ei, pp, lnp, vlng, consts):
    e16, e8, e32, selqk, seld2, zeros_blk = consts
    src = _pad_idx(ei[0], N, EP)
    dst = _pad_idx(ei[1], N, EP)
    src2d = src.reshape(EP // GCH, GCH)
    dst2d = dst.reshape(EP // GCH, GCH)
    dst2d_s = dst.reshape(EP // SCH, SCH)
    zp = jnp.pad(z, ((0, EP - E), (0, 0)))

    td, tsa, tsb = _t1_tables(s, vx, vy, vz, rot9, trans, pp)
    td_g = _sc_gather(td, dst2d, 256, EP)
    tsa_g = _sc_gather(tsa, src2d, 256, EP)
    tsb_g = _sc_gather(tsb, src2d, 384, EP)
    logits, midx = _t2_logits(td_g, tsa_g, zp, dst.reshape(EP, 1), pp,
                              selqk, seld2)
    parts = _sc_segmax(logits.reshape(EP * H), midx.reshape(EP * 16))
    m2 = _t3_mmerge(parts.reshape(NWORK, NT, 8))
    m_g = _sc_gather(m2, dst2d, 128, EP)
    wps = _t4_weights(logits, m_g, tsb_g, zp, e16, e8, e32)
    acc = _sc_scatter(wps, dst2d_s, zeros_blk)
    return _t5_finalize(acc.reshape(2, NPASS, NT, CCOL), s, vx, vy, vz, rot9,
                        trans, inv, pp, lnp, vlng, e16, e8, e32)


def kernel(node_features, rigids_rot, rigids_trans, edge_features, edge_index,
           seq_edge_features, seq_edge_index, x_mask, noising_mask,
           node_vectors, params):
    f32 = jnp.float32
    s0 = node_features
    rot9 = rigids_rot.reshape(N, 9)
    trans = rigids_trans
    inv = (~x_mask).astype(f32).reshape(N, 1)
    noise = noising_mask.reshape(N, 1)
    v_pl = jnp.transpose(node_vectors, (2, 0, 1))
    vx, vy, vz = v_pl[0], v_pl[1], v_pl[2]

    hh = jnp.arange(H)
    e16 = (jnp.arange(128)[None, :] // 16 == hh[:, None]).astype(f32)
    e8 = (jnp.arange(64)[None, :] // 8 == hh[:, None]).astype(f32)
    e32 = (jnp.arange(256)[None, :] // 32 == hh[:, None]).astype(f32)
    selqk = e16.T
    seld2 = ((jnp.arange(96)[:, None] % 32) // 4 == hh[None, :]).astype(f32)
    zeros_blk = jnp.zeros((640, CCOL), f32)
    consts = (e16, e8, e32, selqk, seld2, zeros_blk)

    pA = _prep_psa(params["attn_seq"])
    pB = _prep_psa(params["attn_spatial"])

    s1, vx1, vy1, vz1 = _attention(
        s0, vx, vy, vz, rot9, trans, inv, seq_edge_features, seq_edge_index,
        pA, params["ln_s1"], params["ln_v1"].reshape(1, CV), consts)
    s2, vx2, vy2, vz2 = _attention(
        s1, vx1, vy1, vz1, rot9, trans, inv, edge_features, edge_index,
        pB, params["ln_s2"], params["ln_v2"].reshape(1, CV), consts)

    pr = {
        "lcu": {k: (v.reshape(1, -1) if v.ndim == 1 else v)
                for k, v in params["lcu"].items()},
        "lru": {k: (v.reshape(1, -1) if v.ndim == 1 else v)
                for k, v in params["lru"].items()},
        "nt": {k: (v.reshape(1, -1) if v.ndim == 1 else v)
               for k, v in params["nt"].items()},
        "bb": {"w6": params["bb"]["w6"], "b6": params["bb"]["b6"].reshape(1, 6),
               "wv2": params["bb"]["wv2"]},
        "ln_s3g": params["ln_s3"]["g"].reshape(1, CS),
        "ln_s3b": params["ln_s3"]["b"].reshape(1, CS),
        "ln_v3": params["ln_v3"].reshape(1, CV),
    }
    s3, vfx, vfy, vfz, r9n, trn = _t6_post(s2, vx2, vy2, vz2, rot9, trans,
                                           inv, noise, pr)

    srcB = _pad_idx(edge_index[0], 0, EP)
    dstB = _pad_idx(edge_index[1], 0, EP)
    srcA = _pad_idx(seq_edge_index[0], 0, EP)
    dstA = _pad_idx(seq_edge_index[1], 0, EP)
    idx_et = jnp.concatenate([srcB, dstB, srcA, dstA]).reshape(
        4 * EP // GCH, GCH)
    sg = _sc_gather(s3, idx_et, CS, 4 * EP)

    et = {k: (v.reshape(1, -1) if v.ndim == 1 else v)
          for k, v in params["et"].items()}
    set_ = {k: (v.reshape(1, -1) if v.ndim == 1 else v)
            for k, v in params["set"].items()}
    z_out = _t7_edge_tr(sg, 0, 1, edge_features, et)
    zs_out = _t7_edge_tr(sg, 2, 3, seq_edge_features, set_)

    v_final = jnp.stack([vfx, vfy, vfz], axis=-1)
    return (s3, r9n.reshape(N, 3, 3), trn, z_out, zs_out, v_final)


# speedup vs baseline: 14.1339x; 1.0332x over previous
import functools

import jax
import jax.numpy as jnp
from jax import lax
from jax.experimental import pallas as pl
from jax.experimental.pallas import tpu as pltpu
from jax.experimental.pallas import tpu_sc as plsc

N = 10000
E = 160000
CS = 128
CV = 8
CZ = 32
CH = 16
H = 8
PQ = 4
PV = 8

NT = 10240
EP = 163840
NEG = -3e38
HP = lax.Precision.HIGHEST

NWORK = 32
PER_W = EP // NWORK
GCH = 128
SCH = 64
CCOL = 128
NPASS = 5
ROW_W = 640

@functools.cache
def _mesh():
    return plsc.VectorSubcoreMesh(core_axis_name="c", subcore_axis_name="s")


def _wid():
    return lax.axis_index("s") * 2 + lax.axis_index("c")


def _sc_gather(table, idx2d, D, rows):
    per_w = rows // NWORK
    chunks = per_w // GCH

    @functools.partial(
        pl.kernel,
        out_type=jax.ShapeDtypeStruct((rows, D), jnp.float32),
        mesh=_mesh(),
        scratch_types=[
            pltpu.VMEM((chunks, GCH), jnp.int32),
            pltpu.VMEM((GCH, D), jnp.float32),
            pltpu.SemaphoreType.DMA,
        ],
    )
    def k(table_hbm, idx_hbm, out_hbm, idx_v, rows_v, sem):
        w = _wid()
        row0 = w * per_w
        chunk0 = w * chunks
        pltpu.sync_copy(idx_hbm.at[pl.ds(chunk0, chunks)], idx_v)

        def body(c, _):
            pltpu.async_copy(table_hbm.at[idx_v.at[c]], rows_v, sem).wait()
            pltpu.sync_copy(rows_v, out_hbm.at[pl.ds(row0 + c * GCH, GCH)])
            return 0

        lax.fori_loop(0, chunks, body, 0)

    return k(table, idx2d)


def _sc_segmax(lflat, midx):
    CH_E = 512
    chunks = PER_W // CH_E

    @functools.partial(
        pl.kernel,
        out_type=jax.ShapeDtypeStruct((NWORK, NT * 8), jnp.float32),
        mesh=_mesh(),
        compiler_params=pltpu.CompilerParams(needs_layout_passes=False),
        scratch_types=[
            pltpu.VMEM((NT * 8,), jnp.float32),
            pltpu.VMEM((CH_E * 8,), jnp.float32),
            pltpu.VMEM((CH_E * 16,), jnp.int32),
        ],
    )
    def k(l_hbm, mi_hbm, out_hbm, m_v, l_v, i_v):
        w = _wid()
        base = w * PER_W
        negv = jnp.full((16,), NEG, jnp.float32)
        lanes = lax.iota(jnp.int32, 16)
        lo_mask = lanes < 8

        def init(i, _):
            m_v[pl.ds(i * 16, 16)] = negv
            return 0

        lax.fori_loop(0, NT * 8 // 16, init, 0)

        def chunk(c, _):
            pltpu.sync_copy(l_hbm.at[pl.ds((base + c * CH_E) * 8, CH_E * 8)], l_v)
            pltpu.sync_copy(mi_hbm.at[pl.ds((base + c * CH_E) * 16, CH_E * 16)],
                            i_v)

            def edge(e, _):
                iv = i_v[pl.ds(e * 16, 16)]
                lv = l_v[pl.ds(e * 8, 16)]
                lsel = jnp.where(lo_mask, lv, negv)
                mv = plsc.load_gather(m_v, [iv])
                plsc.store_scatter(m_v, [iv], jnp.maximum(mv, lsel), mask=lo_mask)
                return 0

            lax.fori_loop(0, CH_E, edge, 0)
            return 0

        lax.fori_loop(0, chunks, chunk, 0)
        pltpu.sync_copy(m_v, out_hbm.at[w])

    return k(lflat, midx)


def _sc_scatter(wps, idx2d, zeros_blk):
    chunks = PER_W // SCH

    @functools.partial(
        pl.kernel,
        out_type=jax.ShapeDtypeStruct((2 * NPASS * NT, CCOL), jnp.float32),
        mesh=_mesh(),
        scratch_types=[
            pltpu.VMEM_SHARED((NT, CCOL), jnp.float32),
            pltpu.VMEM((chunks, SCH), jnp.int32),
            pltpu.VMEM((SCH, CCOL), jnp.float32),
            pltpu.SemaphoreType.DMA,
        ],
    )
    def k(w0_h, w1_h, w2_h, w3_h, w4_h, idx_h, z_h, out_h, acc, idx_v, v_buf,
          sem):
        cid = lax.axis_index("c")
        sid = lax.axis_index("s")
        w = sid * 2 + cid
        row0 = w * PER_W
        pltpu.sync_copy(idx_h.at[pl.ds(w * chunks, chunks)], idx_v)
        for p, wp in enumerate((w0_h, w1_h, w2_h, w3_h, w4_h)):
            pltpu.sync_copy(z_h, acc.at[pl.ds(sid * 640, 640)])
            plsc.subcore_barrier()

            def chunk(c, _):
                pltpu.async_copy(wp.at[pl.ds(row0 + c * SCH, SCH)], v_buf, sem).wait()
                pltpu.sync_copy(v_buf, acc.at[idx_v.at[c]], add=True)
                return 0

            lax.fori_loop(0, chunks, chunk, 0)
            plsc.subcore_barrier()
            slot = cid * NPASS + p
            pltpu.sync_copy(acc.at[pl.ds(sid * 640, 640)],
                            out_h.at[pl.ds(slot * NT + sid * 640, 640)])
            plsc.subcore_barrier()

    return k(*wps, idx2d, zeros_blk)


def _sc_mgather(m2flat, midx):
    CH_E = 512
    chunks = PER_W // CH_E

    @functools.partial(
        pl.kernel,
        out_type=jax.ShapeDtypeStruct((EP * 8,), jnp.float32),
        mesh=_mesh(),
        compiler_params=pltpu.CompilerParams(needs_layout_passes=False),
        scratch_types=[
            pltpu.VMEM((NT * 8,), jnp.float32),
            pltpu.VMEM((CH_E * 16,), jnp.int32),
            pltpu.VMEM((CH_E * 8 + 16,), jnp.float32),
        ],
    )
    def k(m_hbm, mi_hbm, out_hbm, m_v, i_v, o_v):
        w = _wid()
        base = w * PER_W
        lanes = lax.iota(jnp.int32, 16)
        lo_mask = lanes < 8

        def tload(t, _):
            pltpu.sync_copy(m_hbm.at[pl.ds(t * 8192, 8192)],
                            m_v.at[pl.ds(t * 8192, 8192)])
            return 0

        lax.fori_loop(0, NT * 8 // 8192, tload, 0)

        def chunk(c, _):
            pltpu.sync_copy(mi_hbm.at[pl.ds((base + c * CH_E) * 16, CH_E * 16)],
                            i_v)

            def edge(e, _):
                iv = i_v[pl.ds(e * 16, 16)]
                mv = plsc.load_gather(m_v, [jnp.where(lo_mask, iv, 0)])
                o_v[pl.ds(e * 8, 16)] = mv
                return 0

            lax.fori_loop(0, CH_E, edge, 0)
            pltpu.sync_copy(o_v.at[pl.ds(0, CH_E * 8)],
                            out_hbm.at[pl.ds((base + c * CH_E) * 8, CH_E * 8)])
            return 0

        lax.fori_loop(0, chunks, chunk, 0)

    return k(m2flat, midx)


def _dot(a, b):
    return jnp.dot(a, b, precision=HP)


def _t1_tables(s, vx, vy, vz, rot9, trans, p):
    B = 512
    grid = NT // B

    def body(s_r, vx_r, vy_r, vz_r, r9_r, tr_r, wq_r, wk_r, wv_r, wqp_r, wkp_r,
             wvp_r, vq_r, vk_r, vv_r, td_r, tsa_r, tsb_r):
        sb = s_r[...]
        vpl = (vx_r[...], vy_r[...], vz_r[...])
        r9 = r9_r[...]
        tr = tr_r[...]
        q = _dot(sb, wq_r[...])
        kk = _dot(sb, wk_r[...])
        vs = _dot(sb, wv_r[...])
        qp = [_dot(sb, wqp_r[j]) + _dot(vpl[j], vq_r[...]) for j in range(3)]
        kp = [_dot(sb, wkp_r[j]) + _dot(vpl[j], vk_r[...]) for j in range(3)]
        vp = [_dot(sb, wvp_r[j]) + _dot(vpl[j], vv_r[...]) for j in range(3)]

        def glob(pts, i):
            return (r9[:, 3 * i:3 * i + 1] * pts[0]
                    + r9[:, 3 * i + 1:3 * i + 2] * pts[1]
                    + r9[:, 3 * i + 2:3 * i + 3] * pts[2]
                    + tr[:, i:i + 1])

        qpg = [glob(qp, i) for i in range(3)]
        kpg = [glob(kp, i) for i in range(3)]
        vpg = [glob(vp, i) for i in range(3)]
        rid = pl.program_id(0) * B + lax.broadcasted_iota(jnp.int32, (B, 1), 0)
        valid = rid < N
        z32 = jnp.zeros((B, 32), jnp.float32)
        td = jnp.where(valid, jnp.concatenate([q] + qpg + [z32], axis=1), 0.0)
        tsa = jnp.where(valid, jnp.concatenate([kk] + kpg + [z32], axis=1), 0.0)
        tsb = jnp.where(valid,
                        jnp.concatenate([vs] + vpg + [z32, z32], axis=1), 0.0)
        td_r[...] = td
        tsa_r[...] = tsa
        tsb_r[...] = tsb

    full = lambda shape: pl.BlockSpec(shape, lambda i: tuple(0 for _ in shape))
    row = lambda w: pl.BlockSpec((B, w), lambda i: (i, 0))
    return pl.pallas_call(
        body,
        grid=(grid,),
        in_specs=[row(CS), row(CV), row(CV), row(CV), row(9), row(3),
                  full((CS, CS)), full((CS, CS)), full((CS, CS)),
                  full((3, CS, H * PQ)), full((3, CS, H * PQ)),
                  full((3, CS, H * PV)),
                  full((CV, H * PQ)), full((CV, H * PQ)), full((CV, H * PV))],
        out_specs=[row(256), row(256), row(384)],
        out_shape=[jax.ShapeDtypeStruct((NT, 256), jnp.float32),
                   jax.ShapeDtypeStruct((NT, 256), jnp.float32),
                   jax.ShapeDtypeStruct((NT, 384), jnp.float32)],
    )(s, vx, vy, vz, rot9, trans, p["wq"], p["wk"], p["wv"], p["wqp3"],
      p["wkp3"], p["wvp3"], p["vq"], p["vk"], p["vv"])


def _t2_logits(td_g, tsa_g, zp, dstcol, p, selqk, seld2):
    B = 512
    grid = EP // B

    def body(td_r, tsa_r, z_r, d_r, wb_r, g_r, sq_r, sd_r, l_r, mi_r):
        td = td_r[...]
        tsa = tsa_r[...]
        z = z_r[...]
        lq = _dot(td[:, :CS] * tsa[:, :CS], sq_r[...]) * 0.25
        d2 = _dot((td[:, CS:224] - tsa[:, CS:224]) ** 2, sd_r[...])
        logits = lq + _dot(z, wb_r[...]) - 0.5 * g_r[...] * d2
        eid = pl.program_id(0) * B + lax.broadcasted_iota(jnp.int32, (B, H), 0)
        l_r[...] = jnp.where(eid < E, logits, NEG)
        mi_r[...] = d_r[...] * 8 + lax.broadcasted_iota(jnp.int32, (B, 16), 1)

    full = lambda shape: pl.BlockSpec(shape, lambda i: tuple(0 for _ in shape))
    row = lambda w: pl.BlockSpec((B, w), lambda i: (i, 0))
    return pl.pallas_call(
        body,
        grid=(grid,),
        in_specs=[row(256), row(256), row(CZ), row(1),
                  full((CZ, H)), full((1, H)), full((CS, H)), full((96, H))],
        out_specs=[row(H), row(16)],
        out_shape=[jax.ShapeDtypeStruct((EP, H), jnp.float32),
                   jax.ShapeDtypeStruct((EP, 16), jnp.int32)],
    )(td_g, tsa_g, zp, dstcol, p["wb"], p["gsp"], selqk, seld2)


def _t3_mmerge(parts):
    B = 512
    grid = NT // B

    def body(p_r, m_r):
        x = p_r[...]
        m = x[0]
        for i in range(1, NWORK):
            m = jnp.maximum(m, x[i])
        m_r[...] = m

    return pl.pallas_call(
        body,
        grid=(grid,),
        in_specs=[pl.BlockSpec((NWORK, B, 8), lambda i: (0, i, 0))],
        out_specs=pl.BlockSpec((B, 8), lambda i: (i, 0)),
        out_shape=jax.ShapeDtypeStruct((NT, 8), jnp.float32),
    )(parts)


def _t4_weights(logits, m_g, tsb_g, zp, e16, e8, e32):
    B = 512
    grid = EP // B

    def body(l_r, m_r, tsb_r, z_r, e16_r, e8_r, e32_r, w0_r, w1_r, w2_r, w3_r,
             w4_r):
        w = jnp.exp(l_r[...] - m_r[...])
        tsb = tsb_r[...]
        z = z_r[...]
        r16 = _dot(w, e16_r[...])
        r8 = _dot(w, e8_r[...])
        r32 = _dot(w, e32_r[...])
        ztile = jnp.concatenate([z] * H, axis=1)
        cat = jnp.concatenate(
            [r16 * tsb[:, :CS]]
            + [r8 * tsb[:, CS + 64 * j:CS + 64 * (j + 1)] for j in range(3)]
            + [r32 * ztile, w, jnp.zeros((B, 56), jnp.float32)], axis=1)
        for i, o_r in enumerate((w0_r, w1_r, w2_r, w3_r, w4_r)):
            o_r[...] = cat[:, i * CCOL:(i + 1) * CCOL]

    full = lambda shape: pl.BlockSpec(shape, lambda i: tuple(0 for _ in shape))
    row = lambda w: pl.BlockSpec((B, w), lambda i: (i, 0))
    return pl.pallas_call(
        body,
        grid=(grid,),
        in_specs=[row(H), row(H), row(384), row(CZ),
                  full((H, 128)), full((H, 64)), full((H, 256))],
        out_specs=[row(CCOL)] * NPASS,
        out_shape=[jax.ShapeDtypeStruct((EP, CCOL), jnp.float32)] * NPASS,
    )(logits, m_g, tsb_g, zp, e16, e8, e32)


def _t5_finalize(acc, s, vx, vy, vz, rot9, trans, inv, p, lnp, vlng,
                 e16, e8, e32):
    B = 400
    grid = N // B

    def body(a_r, s_r, vx_r, vy_r, vz_r, r9_r, tr_r, inv_r, wo_r, bo_r, wpv_r,
             g_r, b_r, vg_r, e16_r, e8_r, e32_r, so_r, vxo_r, vyo_r, vzo_r):
        a = a_r[...]
        acc2 = a[0] + a[1]
        flat = jnp.concatenate([acc2[j] for j in range(NPASS)], axis=1)
        den = flat[:, 576:584] + 1e-9
        d16 = _dot(den, e16_r[...])
        d8 = _dot(den, e8_r[...])
        d32 = _dot(den, e32_r[...])
        o = flat[:, :CS] / d16
        r9 = r9_r[...]
        tr = tr_r[...]
        opt = [flat[:, CS + 64 * j:CS + 64 * (j + 1)] / d8 for j in range(3)]
        optl = [sum((r9[:, 3 * j + i:3 * j + i + 1]
                     * (opt[j] - tr[:, j:j + 1])) for j in range(3))
                for i in range(3)]
        onorm = jnp.sqrt(optl[0] ** 2 + optl[1] ** 2 + optl[2] ** 2 + 1e-8)
        ozn = flat[:, 320:576] / d32
        feats = jnp.concatenate([o] + optl + [onorm, ozn], axis=1)
        su = _dot(feats, wo_r[...]) + bo_r[...]
        inv_b = inv_r[...]
        sn = s_r[...] + su * inv_b
        mu = jnp.mean(sn, axis=1, keepdims=True)
        var = jnp.mean((sn - mu) ** 2, axis=1, keepdims=True)
        so_r[...] = (sn - mu) / jnp.sqrt(var + 1e-5) * g_r[...] + b_r[...]
        vn = [v_r[...] + _dot(optl[i], wpv_r[...]) * inv_b
              for i, v_r in enumerate((vx_r, vy_r, vz_r))]
        n2 = jnp.mean(vn[0] ** 2 + vn[1] ** 2 + vn[2] ** 2, axis=1,
                      keepdims=True)
        scale = vg_r[...] / jnp.sqrt(n2 + 1e-6)
        vxo_r[...] = vn[0] * scale
        vyo_r[...] = vn[1] * scale
        vzo_r[...] = vn[2] * scale

    full = lambda shape: pl.BlockSpec(shape, lambda i: tuple(0 for _ in shape))
    row = lambda w: pl.BlockSpec((B, w), lambda i: (i, 0))
    return pl.pallas_call(
        body,
        grid=(grid,),
        in_specs=[pl.BlockSpec((2, NPASS, B, CCOL), lambda i: (0, 0, i, 0)),
                  row(CS), row(CV), row(CV), row(CV), row(9), row(3), row(1),
                  full((640, CS)), full((1, CS)), full((64, CV)),
                  full((1, CS)), full((1, CS)), full((1, CV)),
                  full((H, 128)), full((H, 64)), full((H, 256))],
        out_specs=[row(CS), row(CV), row(CV), row(CV)],
        out_shape=[jax.ShapeDtypeStruct((N, CS), jnp.float32)]
        + [jax.ShapeDtypeStruct((N, CV), jnp.float32)] * 3,
    )(acc, s, vx, vy, vz, rot9, trans, inv, p["woP"], p["bo"], p["wpv"],
      lnp["g"].reshape(1, CS), lnp["b"].reshape(1, CS), vlng, e16, e8, e32)


def _t6_post(s, vx, vy, vz, rot9, trans, inv, noise, pr):
    B = 400
    grid = N // B

    def body(s_r, vx_r, vy_r, vz_r, r9_r, tr_r, inv_r, no_r,
             lc_w1, lc_b1, lc_ws, lc_bs, lc_wg, lc_bg, lc_wm,
             lr_w1, lr_b1, lr_ws, lr_bs, lr_wg, lr_bg, lr_wm,
             l3g, l3b, v3g,
             nt_w1, nt_b1, nt_w2, nt_b2, nt_g, nt_b, nt_wg, nt_bg, nt_wm,
             w6_r, b6_r, wv2_r,
             so_r, vxo_r, vyo_r, vzo_r, r9o_r, tro_r):
        s_ = s_r[...]
        v = [vx_r[...], vy_r[...], vz_r[...]]
        inv_b = inv_r[...]
        no_b = no_r[...]

        def gate(s_, v, w1, b1, ws, bs, wg, bg, wm, act):
            nrm = jnp.sqrt(v[0] ** 2 + v[1] ** 2 + v[2] ** 2 + 1e-8)
            h = jnp.maximum(_dot(jnp.concatenate([s_, nrm], axis=1), w1[...])
                            + b1[...], 0.0)
            su = _dot(h, ws[...]) + bs[...]
            g = act(_dot(h, wg[...]) + bg[...])
            vu = [g * _dot(v[i], wm[...]) for i in range(3)]
            return su, vu

        su, vu = gate(s_, v, lc_w1, lc_b1, lc_ws, lc_bs, lc_wg, lc_bg, lc_wm,
                      jax.nn.sigmoid)
        s_ = s_ + su * inv_b
        v = [v[i] + vu[i] * inv_b for i in range(3)]
        su, vu = gate(s_, v, lr_w1, lr_b1, lr_ws, lr_bs, lr_wg, lr_bg, lr_wm,
                      jnp.tanh)
        sn = s_ + su * inv_b

        def ln(x, g, b):
            mu = jnp.mean(x, axis=1, keepdims=True)
            var = jnp.mean((x - mu) ** 2, axis=1, keepdims=True)
            return (x - mu) / jnp.sqrt(var + 1e-5) * g[...] + b[...]

        s_ = ln(sn, l3g, l3b)
        v = [v[i] + vu[i] * inv_b for i in range(3)]
        n2 = jnp.mean(v[0] ** 2 + v[1] ** 2 + v[2] ** 2, axis=1, keepdims=True)
        v = [v[i] * (v3g[...] / jnp.sqrt(n2 + 1e-6)) for i in range(3)]

        h = jnp.maximum(_dot(s_, nt_w1[...]) + nt_b1[...], 0.0)
        s2 = ln(s_ + _dot(h, nt_w2[...]) + nt_b2[...], nt_g, nt_b)
        gg = jax.nn.sigmoid(_dot(s2, nt_wg[...]) + nt_bg[...])
        v = [v[i] + gg * _dot(v[i], nt_wm[...]) for i in range(3)]

        s_f = s2 * inv_b
        v_f = [v[i] * inv_b for i in range(3)]

        sb = s_f * no_b
        vb = [v_f[i] * no_b for i in range(3)]
        a = [_dot(vb[i], wv2_r[...]) for i in range(3)]
        vc = jnp.concatenate([a[0][:, :1], a[1][:, :1], a[2][:, :1],
                              a[0][:, 1:], a[1][:, 1:], a[2][:, 1:]], axis=1)
        upd = (_dot(sb, w6_r[...]) + b6_r[...] + vc) * no_b

        qn = jnp.sqrt(1.0 + upd[:, 0:1] ** 2 + upd[:, 1:2] ** 2
                      + upd[:, 2:3] ** 2)
        qw = 1.0 / qn
        qx = upd[:, 0:1] / qn
        qy = upd[:, 1:2] / qn
        qz = upd[:, 2:3] / qn
        ru = [1 - 2 * (qy * qy + qz * qz), 2 * (qx * qy - qz * qw),
              2 * (qx * qz + qy * qw),
              2 * (qx * qy + qz * qw), 1 - 2 * (qx * qx + qz * qz),
              2 * (qy * qz - qx * qw),
              2 * (qx * qz - qy * qw), 2 * (qy * qz + qx * qw),
              1 - 2 * (qx * qx + qy * qy)]
        r9 = r9_r[...]
        newr = [sum(r9[:, 3 * i + j:3 * i + j + 1] * ru[3 * j + k]
                    for j in range(3)) for i in range(3) for k in range(3)]
        tr = tr_r[...]
        newt = [tr[:, i:i + 1]
                + sum(r9[:, 3 * i + j:3 * i + j + 1] * upd[:, 3 + j:4 + j]
                      for j in range(3)) for i in range(3)]
        so_r[...] = s_f
        vxo_r[...] = v_f[0]
        vyo_r[...] = v_f[1]
        vzo_r[...] = v_f[2]
        r9o_r[...] = jnp.concatenate(newr, axis=1)
        tro_r[...] = jnp.concatenate(newt, axis=1)

    full = lambda shape: pl.BlockSpec(shape, lambda i: tuple(0 for _ in shape))
    row = lambda w: pl.BlockSpec((B, w), lambda i: (i, 0))
    lcu, lru, nt, bb = pr["lcu"], pr["lru"], pr["nt"], pr["bb"]
    return pl.pallas_call(
        body,
        grid=(grid,),
        in_specs=[row(CS), row(CV), row(CV), row(CV), row(9), row(3), row(1),
                  row(1),
                  full((CS + CV, CS)), full((1, CS)), full((CS, CS)),
                  full((1, CS)), full((CS, CV)), full((1, CV)), full((CV, CV)),
                  full((CS + CV, CS)), full((1, CS)), full((CS, CS)),
                  full((1, CS)), full((CS, CV)), full((1, CV)), full((CV, CV)),
                  full((1, CS)), full((1, CS)), full((1, CV)),
                  full((CS, 2 * CS)), full((1, 2 * CS)), full((2 * CS, CS)),
                  full((1, CS)), full((1, CS)), full((1, CS)),
                  full((CS, CV)), full((1, CV)), full((CV, CV)),
                  full((CS, 6)), full((1, 6)), full((CV, 2))],
        out_specs=[row(CS), row(CV), row(CV), row(CV), row(9), row(3)],
        out_shape=[jax.ShapeDtypeStruct((N, CS), jnp.float32),
                   jax.ShapeDtypeStruct((N, CV), jnp.float32),
                   jax.ShapeDtypeStruct((N, CV), jnp.float32),
                   jax.ShapeDtypeStruct((N, CV), jnp.float32),
                   jax.ShapeDtypeStruct((N, 9), jnp.float32),
                   jax.ShapeDtypeStruct((N, 3), jnp.float32)],
    )(s, vx, vy, vz, rot9, trans, inv, noise,
      lcu["w1"], lcu["b1"], lcu["ws"], lcu["bs"], lcu["wg"], lcu["bg"],
      lcu["wm"],
      lru["w1"], lru["b1"], lru["ws"], lru["bs"], lru["wg"], lru["bg"],
      lru["wm"],
      pr["ln_s3g"], pr["ln_s3b"], pr["ln_v3"],
      nt["w1"], nt["b1"], nt["w2"], nt["b2"], nt["g"], nt["b"], nt["wg"],
      nt["bg"], nt["wm"],
      bb["w6"], bb["b6"], bb["wv2"])


def _t7_edge_tr(sg, piece_src, piece_dst, z, p):
    B = 512
    grid = (E + B - 1) // B
    off_s = piece_src * (EP // B)
    off_d = piece_dst * (EP // B)

    def body(ss_r, sd_r, z_r, w1_r, b1_r, w2_r, b2_r, g_r, b_r, zo_r):
        z_ = z_r[...]
        hcat = jnp.concatenate([ss_r[...], sd_r[...], z_], axis=1)
        h = jnp.maximum(_dot(hcat, w1_r[...]) + b1_r[...], 0.0)
        zn = z_ + _dot(h, w2_r[...]) + b2_r[...]
        mu = jnp.mean(zn, axis=1, keepdims=True)
        var = jnp.mean((zn - mu) ** 2, axis=1, keepdims=True)
        zo_r[...] = (zn - mu) / jnp.sqrt(var + 1e-5) * g_r[...] + b_r[...]

    full = lambda shape: pl.BlockSpec(shape, lambda i: tuple(0 for _ in shape))
    return pl.pallas_call(
        body,
        grid=(grid,),
        in_specs=[pl.BlockSpec((B, CS), lambda i: (i + off_s, 0)),
                  pl.BlockSpec((B, CS), lambda i: (i + off_d, 0)),
                  pl.BlockSpec((B, CZ), lambda i: (i, 0)),
                  full((2 * CS + CZ, 2 * CZ)), full((1, 2 * CZ)),
                  full((2 * CZ, CZ)), full((1, CZ)),
                  full((1, CZ)), full((1, CZ))],
        out_specs=pl.BlockSpec((B, CZ), lambda i: (i, 0)),
        out_shape=jax.ShapeDtypeStruct((E, CZ), jnp.float32),
    )(sg, sg, z, p["w1"], p["b1"], p["w2"], p["b2"], p["g"], p["b"])


def _prep_psa(p):
    wo = p["wo"]
    o_part = wo[:CS]
    optl_part = wo[CS:CS + 192].reshape(64, 3, CS).transpose(1, 0, 2).reshape(192, CS)
    rest = wo[CS + 192:]
    return {
        "wq": p["wq"], "wk": p["wk"], "wv": p["wv"],
        "wqp3": p["wqp"].reshape(CS, H * PQ, 3).transpose(2, 0, 1),
        "wkp3": p["wkp"].reshape(CS, H * PQ, 3).transpose(2, 0, 1),
        "wvp3": p["wvp"].reshape(CS, H * PV, 3).transpose(2, 0, 1),
        "vq": p["vq"], "vk": p["vk"], "vv": p["vv"],
        "wb": p["wb"],
        "gsp": jax.nn.softplus(p["gamma"]).reshape(1, H),
        "woP": jnp.concatenate([o_part, optl_part, rest], axis=0),
        "bo": p["bo"].reshape(1, CS),
        "wpv": p["wpv"],
    }


def _pad_idx(idx, pad_val, rows):
    return jnp.pad(idx, (0, rows - idx.shape[0]), constant_values=pad_val)


def _attention(s, vx, vy, vz, rot9, trans, inv, z, ei, pp, lnp, vlng, consts):
    e16, e8, e32, selqk, seld2, zeros_blk = consts
    src = _pad_idx(ei[0], N, EP)
    dst = _pad_idx(ei[1], N, EP)
    src2d = src.reshape(EP // GCH, GCH)
    dst2d = dst.reshape(EP // GCH, GCH)
    dst2d_s = dst.reshape(EP // SCH, SCH)
    zp = jnp.pad(z, ((0, EP - E), (0, 0)))

    td, tsa, tsb = _t1_tables(s, vx, vy, vz, rot9, trans, pp)
    td_g = _sc_gather(td, dst2d, 256, EP)
    tsa_g = _sc_gather(tsa, src2d, 256, EP)
    tsb_g = _sc_gather(tsb, src2d, 384, EP)
    logits, midx = _t2_logits(td_g, tsa_g, zp, dst.reshape(EP, 1), pp,
                              selqk, seld2)
    midx_f = midx.reshape(EP * 16)
    parts = _sc_segmax(logits.reshape(EP * H), midx_f)
    m2 = _t3_mmerge(parts.reshape(NWORK, NT, 8))
    m_g = _sc_mgather(m2.reshape(NT * 8), midx_f).reshape(EP, 8)
    wps = _t4_weights(logits, m_g, tsb_g, zp, e16, e8, e32)
    acc = _sc_scatter(wps, dst2d_s, zeros_blk)
    return _t5_finalize(acc.reshape(2, NPASS, NT, CCOL), s, vx, vy, vz, rot9,
                        trans, inv, pp, lnp, vlng, e16, e8, e32)


def kernel(node_features, rigids_rot, rigids_trans, edge_features, edge_index,
           seq_edge_features, seq_edge_index, x_mask, noising_mask,
           node_vectors, params):
    f32 = jnp.float32
    s0 = node_features
    rot9 = rigids_rot.reshape(N, 9)
    trans = rigids_trans
    inv = (~x_mask).astype(f32).reshape(N, 1)
    noise = noising_mask.reshape(N, 1)
    v_pl = jnp.transpose(node_vectors, (2, 0, 1))
    vx, vy, vz = v_pl[0], v_pl[1], v_pl[2]

    hh = jnp.arange(H)
    e16 = (jnp.arange(128)[None, :] // 16 == hh[:, None]).astype(f32)
    e8 = (jnp.arange(64)[None, :] // 8 == hh[:, None]).astype(f32)
    e32 = (jnp.arange(256)[None, :] // 32 == hh[:, None]).astype(f32)
    selqk = e16.T
    seld2 = ((jnp.arange(96)[:, None] % 32) // 4 == hh[None, :]).astype(f32)
    zeros_blk = jnp.zeros((640, CCOL), f32)
    consts = (e16, e8, e32, selqk, seld2, zeros_blk)

    pA = _prep_psa(params["attn_seq"])
    pB = _prep_psa(params["attn_spatial"])

    s1, vx1, vy1, vz1 = _attention(
        s0, vx, vy, vz, rot9, trans, inv, seq_edge_features, seq_edge_index,
        pA, params["ln_s1"], params["ln_v1"].reshape(1, CV), consts)
    s2, vx2, vy2, vz2 = _attention(
        s1, vx1, vy1, vz1, rot9, trans, inv, edge_features, edge_index,
        pB, params["ln_s2"], params["ln_v2"].reshape(1, CV), consts)

    pr = {
        "lcu": {k: (v.reshape(1, -1) if v.ndim == 1 else v)
                for k, v in params["lcu"].items()},
        "lru": {k: (v.reshape(1, -1) if v.ndim == 1 else v)
                for k, v in params["lru"].items()},
        "nt": {k: (v.reshape(1, -1) if v.ndim == 1 else v)
               for k, v in params["nt"].items()},
        "bb": {"w6": params["bb"]["w6"], "b6": params["bb"]["b6"].reshape(1, 6),
               "wv2": params["bb"]["wv2"]},
        "ln_s3g": params["ln_s3"]["g"].reshape(1, CS),
        "ln_s3b": params["ln_s3"]["b"].reshape(1, CS),
        "ln_v3": params["ln_v3"].reshape(1, CV),
    }
    s3, vfx, vfy, vfz, r9n, trn = _t6_post(s2, vx2, vy2, vz2, rot9, trans,
                                           inv, noise, pr)

    srcB = _pad_idx(edge_index[0], 0, EP)
    dstB = _pad_idx(edge_index[1], 0, EP)
    srcA = _pad_idx(seq_edge_index[0], 0, EP)
    dstA = _pad_idx(seq_edge_index[1], 0, EP)
    idx_et = jnp.concatenate([srcB, dstB, srcA, dstA]).reshape(
        4 * EP // GCH, GCH)
    sg = _sc_gather(s3, idx_et, CS, 4 * EP)

    et = {k: (v.reshape(1, -1) if v.ndim == 1 else v)
          for k, v in params["et"].items()}
    set_ = {k: (v.reshape(1, -1) if v.ndim == 1 else v)
            for k, v in params["set"].items()}
    z_out = _t7_edge_tr(sg, 0, 1, edge_features, et)
    zs_out = _t7_edge_tr(sg, 2, 3, seq_edge_features, set_)

    v_final = jnp.stack([vfx, vfy, vfz], axis=-1)
    return (s3, r9n.reshape(N, 3, 3), trn, z_out, zs_out, v_final)


# speedup vs baseline: 15.4321x; 1.0919x over previous
import functools

import jax
import jax.numpy as jnp
from jax import lax
from jax.experimental import pallas as pl
from jax.experimental.pallas import tpu as pltpu
from jax.experimental.pallas import tpu_sc as plsc

N = 10000
E = 160000
CS = 128
CV = 8
CZ = 32
CH = 16
H = 8
PQ = 4
PV = 8

NT = 10240
EP = 163840
NEG = -3e38
HP = lax.Precision.HIGHEST

NWORK = 32
PER_W = EP // NWORK
GCH = 128
SCH = 128
CCOL = 128
NPASS = 5
ROW_W = 640

@functools.cache
def _mesh():
    return plsc.VectorSubcoreMesh(core_axis_name="c", subcore_axis_name="s")


def _wid():
    return lax.axis_index("s") * 2 + lax.axis_index("c")


def _sc_gather(table, idx2d, D, rows):
    per_w = rows // NWORK
    chunks = per_w // GCH

    @functools.partial(
        pl.kernel,
        out_type=jax.ShapeDtypeStruct((rows, D), jnp.float32),
        mesh=_mesh(),
        scratch_types=[
            pltpu.VMEM((chunks, GCH), jnp.int32),
            pltpu.VMEM((2, GCH, D), jnp.float32),
            pltpu.SemaphoreType.DMA,
            pltpu.SemaphoreType.DMA,
        ],
    )
    def k(table_hbm, idx_hbm, out_hbm, idx_v, rows_v, sem0, sem1):
        w = _wid()
        row0 = w * per_w
        chunk0 = w * chunks
        sems = (sem0, sem1)
        pltpu.sync_copy(idx_hbm.at[pl.ds(chunk0, chunks)], idx_v)
        for b in range(2):
            pltpu.async_copy(table_hbm.at[idx_v.at[b]], rows_v.at[b], sems[b])

        def pair(g, _):
            for b in range(2):
                c = g * 2 + b
                pltpu.make_async_copy(table_hbm.at[idx_v.at[c]],
                                      rows_v.at[b], sems[b]).wait()
                pltpu.sync_copy(rows_v.at[b],
                                out_hbm.at[pl.ds(row0 + c * GCH, GCH)])
                cn = jnp.minimum(c + 2, chunks - 1)
                pltpu.async_copy(table_hbm.at[idx_v.at[cn]], rows_v.at[b],
                                 sems[b])
            return 0

        lax.fori_loop(0, chunks // 2, pair, 0)
        for b in range(2):
            pltpu.make_async_copy(table_hbm.at[idx_v.at[chunks - 1]],
                                  rows_v.at[b], sems[b]).wait()

    return k(table, idx2d)


def _sc_segmax(lflat, midx):
    CH_E = 512
    chunks = PER_W // CH_E

    @functools.partial(
        pl.kernel,
        out_type=jax.ShapeDtypeStruct((NWORK, NT * 8), jnp.float32),
        mesh=_mesh(),
        compiler_params=pltpu.CompilerParams(needs_layout_passes=False),
        scratch_types=[
            pltpu.VMEM((NT * 8,), jnp.float32),
            pltpu.VMEM((CH_E * 8,), jnp.float32),
            pltpu.VMEM((CH_E * 16,), jnp.int32),
        ],
    )
    def k(l_hbm, mi_hbm, out_hbm, m_v, l_v, i_v):
        w = _wid()
        base = w * PER_W
        negv = jnp.full((16,), NEG, jnp.float32)
        lanes = lax.iota(jnp.int32, 16)
        lo_mask = lanes < 8

        def init(i, _):
            m_v[pl.ds(i * 16, 16)] = negv
            return 0

        lax.fori_loop(0, NT * 8 // 16, init, 0)

        def chunk(c, _):
            pltpu.sync_copy(l_hbm.at[pl.ds((base + c * CH_E) * 8, CH_E * 8)], l_v)
            pltpu.sync_copy(mi_hbm.at[pl.ds((base + c * CH_E) * 16, CH_E * 16)],
                            i_v)

            def edge(e, _):
                iv = i_v[pl.ds(e * 16, 16)]
                lv = l_v[pl.ds(e * 8, 16)]
                lsel = jnp.where(lo_mask, lv, negv)
                mv = plsc.load_gather(m_v, [iv])
                plsc.store_scatter(m_v, [iv], jnp.maximum(mv, lsel), mask=lo_mask)
                return 0

            lax.fori_loop(0, CH_E, edge, 0)
            return 0

        lax.fori_loop(0, chunks, chunk, 0)
        pltpu.sync_copy(m_v, out_hbm.at[w])

    return k(lflat, midx)


def _sc_scatter(wps, idx2d, zeros_blk):
    chunks = PER_W // SCH

    @functools.partial(
        pl.kernel,
        out_type=jax.ShapeDtypeStruct((2 * NPASS * NT, CCOL), jnp.float32),
        mesh=_mesh(),
        scratch_types=[
            pltpu.VMEM_SHARED((NT, CCOL), jnp.float32),
            pltpu.VMEM((chunks, SCH), jnp.int32),
            pltpu.VMEM((2, SCH, CCOL), jnp.float32),
            pltpu.SemaphoreType.DMA,
            pltpu.SemaphoreType.DMA,
        ],
    )
    def k(w0_h, w1_h, w2_h, w3_h, w4_h, idx_h, z_h, out_h, acc, idx_v, v_buf,
          sem0, sem1):
        cid = lax.axis_index("c")
        sid = lax.axis_index("s")
        w = sid * 2 + cid
        row0 = w * PER_W
        sems = (sem0, sem1)
        pltpu.sync_copy(idx_h.at[pl.ds(w * chunks, chunks)], idx_v)
        for p, wp in enumerate((w0_h, w1_h, w2_h, w3_h, w4_h)):
            pltpu.sync_copy(z_h, acc.at[pl.ds(sid * 640, 640)])
            plsc.subcore_barrier()
            for b in range(2):
                pltpu.async_copy(wp.at[pl.ds(row0 + b * SCH, SCH)],
                                 v_buf.at[b], sems[b])

            def pair(g, _):
                for b in range(2):
                    c = g * 2 + b
                    pltpu.make_async_copy(wp.at[pl.ds(row0 + c * SCH, SCH)],
                                          v_buf.at[b], sems[b]).wait()
                    pltpu.sync_copy(v_buf.at[b], acc.at[idx_v.at[c]], add=True)
                    cn = jnp.minimum(c + 2, chunks - 1)
                    pltpu.async_copy(wp.at[pl.ds(row0 + cn * SCH, SCH)],
                                     v_buf.at[b], sems[b])
                return 0

            lax.fori_loop(0, chunks // 2, pair, 0)
            for b in range(2):
                pltpu.make_async_copy(
                    wp.at[pl.ds(row0 + (chunks - 1) * SCH, SCH)],
                    v_buf.at[b], sems[b]).wait()
            plsc.subcore_barrier()
            slot = cid * NPASS + p
            pltpu.sync_copy(acc.at[pl.ds(sid * 640, 640)],
                            out_h.at[pl.ds(slot * NT + sid * 640, 640)])
            plsc.subcore_barrier()

    return k(*wps, idx2d, zeros_blk)


def _sc_mgather(m2flat, midx):
    CH_E = 512
    chunks = PER_W // CH_E

    @functools.partial(
        pl.kernel,
        out_type=jax.ShapeDtypeStruct((EP * 8,), jnp.float32),
        mesh=_mesh(),
        compiler_params=pltpu.CompilerParams(needs_layout_passes=False),
        scratch_types=[
            pltpu.VMEM((NT * 8,), jnp.float32),
            pltpu.VMEM((CH_E * 16,), jnp.int32),
            pltpu.VMEM((CH_E * 8 + 16,), jnp.float32),
        ],
    )
    def k(m_hbm, mi_hbm, out_hbm, m_v, i_v, o_v):
        w = _wid()
        base = w * PER_W
        lanes = lax.iota(jnp.int32, 16)
        lo_mask = lanes < 8

        def tload(t, _):
            pltpu.sync_copy(m_hbm.at[pl.ds(t * 8192, 8192)],
                            m_v.at[pl.ds(t * 8192, 8192)])
            return 0

        lax.fori_loop(0, NT * 8 // 8192, tload, 0)

        def chunk(c, _):
            pltpu.sync_copy(mi_hbm.at[pl.ds((base + c * CH_E) * 16, CH_E * 16)],
                            i_v)

            def edge(e, _):
                iv = i_v[pl.ds(e * 16, 16)]
                mv = plsc.load_gather(m_v, [jnp.where(lo_mask, iv, 0)])
                o_v[pl.ds(e * 8, 16)] = mv
                return 0

            lax.fori_loop(0, CH_E, edge, 0)
            pltpu.sync_copy(o_v.at[pl.ds(0, CH_E * 8)],
                            out_hbm.at[pl.ds((base + c * CH_E) * 8, CH_E * 8)])
            return 0

        lax.fori_loop(0, chunks, chunk, 0)

    return k(m2flat, midx)


def _dot(a, b):
    return jnp.dot(a, b, precision=HP)


def _t1_tables(s, vx, vy, vz, rot9, trans, p):
    B = 512
    grid = NT // B

    def body(s_r, vx_r, vy_r, vz_r, r9_r, tr_r, wq_r, wk_r, wv_r, wqp_r, wkp_r,
             wvp_r, vq_r, vk_r, vv_r, td_r, tsa_r, tsb_r):
        sb = s_r[...]
        vpl = (vx_r[...], vy_r[...], vz_r[...])
        r9 = r9_r[...]
        tr = tr_r[...]
        q = _dot(sb, wq_r[...])
        kk = _dot(sb, wk_r[...])
        vs = _dot(sb, wv_r[...])
        qp = [_dot(sb, wqp_r[j]) + _dot(vpl[j], vq_r[...]) for j in range(3)]
        kp = [_dot(sb, wkp_r[j]) + _dot(vpl[j], vk_r[...]) for j in range(3)]
        vp = [_dot(sb, wvp_r[j]) + _dot(vpl[j], vv_r[...]) for j in range(3)]

        def glob(pts, i):
            return (r9[:, 3 * i:3 * i + 1] * pts[0]
                    + r9[:, 3 * i + 1:3 * i + 2] * pts[1]
                    + r9[:, 3 * i + 2:3 * i + 3] * pts[2]
                    + tr[:, i:i + 1])

        qpg = [glob(qp, i) for i in range(3)]
        kpg = [glob(kp, i) for i in range(3)]
        vpg = [glob(vp, i) for i in range(3)]
        rid = pl.program_id(0) * B + lax.broadcasted_iota(jnp.int32, (B, 1), 0)
        valid = rid < N
        z32 = jnp.zeros((B, 32), jnp.float32)
        td = jnp.where(valid, jnp.concatenate([q] + qpg + [z32], axis=1), 0.0)
        tsa = jnp.where(valid, jnp.concatenate([kk] + kpg + [z32], axis=1), 0.0)
        tsb = jnp.where(valid,
                        jnp.concatenate([vs] + vpg + [z32, z32], axis=1), 0.0)
        td_r[...] = td
        tsa_r[...] = tsa
        tsb_r[...] = tsb

    full = lambda shape: pl.BlockSpec(shape, lambda i: tuple(0 for _ in shape))
    row = lambda w: pl.BlockSpec((B, w), lambda i: (i, 0))
    return pl.pallas_call(
        body,
        grid=(grid,),
        in_specs=[row(CS), row(CV), row(CV), row(CV), row(9), row(3),
                  full((CS, CS)), full((CS, CS)), full((CS, CS)),
                  full((3, CS, H * PQ)), full((3, CS, H * PQ)),
                  full((3, CS, H * PV)),
                  full((CV, H * PQ)), full((CV, H * PQ)), full((CV, H * PV))],
        out_specs=[row(256), row(256), row(384)],
        out_shape=[jax.ShapeDtypeStruct((NT, 256), jnp.float32),
                   jax.ShapeDtypeStruct((NT, 256), jnp.float32),
                   jax.ShapeDtypeStruct((NT, 384), jnp.float32)],
    )(s, vx, vy, vz, rot9, trans, p["wq"], p["wk"], p["wv"], p["wqp3"],
      p["wkp3"], p["wvp3"], p["vq"], p["vk"], p["vv"])


def _t2_logits(td_g, tsa_g, zp, dstcol, p, selqk, seld2):
    B = 512
    grid = EP // B

    def body(td_r, tsa_r, z_r, d_r, wb_r, g_r, sq_r, sd_r, l_r, mi_r):
        td = td_r[...]
        tsa = tsa_r[...]
        z = z_r[...]
        lq = _dot(td[:, :CS] * tsa[:, :CS], sq_r[...]) * 0.25
        d2 = _dot((td[:, CS:224] - tsa[:, CS:224]) ** 2, sd_r[...])
        logits = lq + _dot(z, wb_r[...]) - 0.5 * g_r[...] * d2
        eid = pl.program_id(0) * B + lax.broadcasted_iota(jnp.int32, (B, H), 0)
        l_r[...] = jnp.where(eid < E, logits, NEG)
        mi_r[...] = d_r[...] * 8 + lax.broadcasted_iota(jnp.int32, (B, 16), 1)

    full = lambda shape: pl.BlockSpec(shape, lambda i: tuple(0 for _ in shape))
    row = lambda w: pl.BlockSpec((B, w), lambda i: (i, 0))
    return pl.pallas_call(
        body,
        grid=(grid,),
        in_specs=[row(256), row(256), row(CZ), row(1),
                  full((CZ, H)), full((1, H)), full((CS, H)), full((96, H))],
        out_specs=[row(H), row(16)],
        out_shape=[jax.ShapeDtypeStruct((EP, H), jnp.float32),
                   jax.ShapeDtypeStruct((EP, 16), jnp.int32)],
    )(td_g, tsa_g, zp, dstcol, p["wb"], p["gsp"], selqk, seld2)


def _t3_mmerge(parts):
    B = 512
    grid = NT // B

    def body(p_r, m_r):
        x = p_r[...]
        m = x[0]
        for i in range(1, NWORK):
            m = jnp.maximum(m, x[i])
        m_r[...] = m

    return pl.pallas_call(
        body,
        grid=(grid,),
        in_specs=[pl.BlockSpec((NWORK, B, 8), lambda i: (0, i, 0))],
        out_specs=pl.BlockSpec((B, 8), lambda i: (i, 0)),
        out_shape=jax.ShapeDtypeStruct((NT, 8), jnp.float32),
    )(parts)


def _t4_weights(logits, m_g, tsb_g, zp, e16, e8, e32):
    B = 512
    grid = EP // B

    def body(l_r, m_r, tsb_r, z_r, e16_r, e8_r, e32_r, w0_r, w1_r, w2_r, w3_r,
             w4_r):
        w = jnp.exp(l_r[...] - m_r[...])
        tsb = tsb_r[...]
        z = z_r[...]
        r16 = _dot(w, e16_r[...])
        r8 = _dot(w, e8_r[...])
        r32 = _dot(w, e32_r[...])
        ztile = jnp.concatenate([z] * H, axis=1)
        cat = jnp.concatenate(
            [r16 * tsb[:, :CS]]
            + [r8 * tsb[:, CS + 64 * j:CS + 64 * (j + 1)] for j in range(3)]
            + [r32 * ztile, w, jnp.zeros((B, 56), jnp.float32)], axis=1)
        for i, o_r in enumerate((w0_r, w1_r, w2_r, w3_r, w4_r)):
            o_r[...] = cat[:, i * CCOL:(i + 1) * CCOL]

    full = lambda shape: pl.BlockSpec(shape, lambda i: tuple(0 for _ in shape))
    row = lambda w: pl.BlockSpec((B, w), lambda i: (i, 0))
    return pl.pallas_call(
        body,
        grid=(grid,),
        in_specs=[row(H), row(H), row(384), row(CZ),
                  full((H, 128)), full((H, 64)), full((H, 256))],
        out_specs=[row(CCOL)] * NPASS,
        out_shape=[jax.ShapeDtypeStruct((EP, CCOL), jnp.float32)] * NPASS,
    )(logits, m_g, tsb_g, zp, e16, e8, e32)


def _t5_finalize(acc, s, vx, vy, vz, rot9, trans, inv, p, lnp, vlng,
                 e16, e8, e32):
    B = 400
    grid = N // B

    def body(a_r, s_r, vx_r, vy_r, vz_r, r9_r, tr_r, inv_r, wo_r, bo_r, wpv_r,
             g_r, b_r, vg_r, e16_r, e8_r, e32_r, so_r, vxo_r, vyo_r, vzo_r):
        a = a_r[...]
        acc2 = a[0] + a[1]
        flat = jnp.concatenate([acc2[j] for j in range(NPASS)], axis=1)
        den = flat[:, 576:584] + 1e-9
        d16 = _dot(den, e16_r[...])
        d8 = _dot(den, e8_r[...])
        d32 = _dot(den, e32_r[...])
        o = flat[:, :CS] / d16
        r9 = r9_r[...]
        tr = tr_r[...]
        opt = [flat[:, CS + 64 * j:CS + 64 * (j + 1)] / d8 for j in range(3)]
        optl = [sum((r9[:, 3 * j + i:3 * j + i + 1]
                     * (opt[j] - tr[:, j:j + 1])) for j in range(3))
                for i in range(3)]
        onorm = jnp.sqrt(optl[0] ** 2 + optl[1] ** 2 + optl[2] ** 2 + 1e-8)
        ozn = flat[:, 320:576] / d32
        feats = jnp.concatenate([o] + optl + [onorm, ozn], axis=1)
        su = _dot(feats, wo_r[...]) + bo_r[...]
        inv_b = inv_r[...]
        sn = s_r[...] + su * inv_b
        mu = jnp.mean(sn, axis=1, keepdims=True)
        var = jnp.mean((sn - mu) ** 2, axis=1, keepdims=True)
        so_r[...] = (sn - mu) / jnp.sqrt(var + 1e-5) * g_r[...] + b_r[...]
        vn = [v_r[...] + _dot(optl[i], wpv_r[...]) * inv_b
              for i, v_r in enumerate((vx_r, vy_r, vz_r))]
        n2 = jnp.mean(vn[0] ** 2 + vn[1] ** 2 + vn[2] ** 2, axis=1,
                      keepdims=True)
        scale = vg_r[...] / jnp.sqrt(n2 + 1e-6)
        vxo_r[...] = vn[0] * scale
        vyo_r[...] = vn[1] * scale
        vzo_r[...] = vn[2] * scale

    full = lambda shape: pl.BlockSpec(shape, lambda i: tuple(0 for _ in shape))
    row = lambda w: pl.BlockSpec((B, w), lambda i: (i, 0))
    return pl.pallas_call(
        body,
        grid=(grid,),
        in_specs=[pl.BlockSpec((2, NPASS, B, CCOL), lambda i: (0, 0, i, 0)),
                  row(CS), row(CV), row(CV), row(CV), row(9), row(3), row(1),
                  full((640, CS)), full((1, CS)), full((64, CV)),
                  full((1, CS)), full((1, CS)), full((1, CV)),
                  full((H, 128)), full((H, 64)), full((H, 256))],
        out_specs=[row(CS), row(CV), row(CV), row(CV)],
        out_shape=[jax.ShapeDtypeStruct((N, CS), jnp.float32)]
        + [jax.ShapeDtypeStruct((N, CV), jnp.float32)] * 3,
    )(acc, s, vx, vy, vz, rot9, trans, inv, p["woP"], p["bo"], p["wpv"],
      lnp["g"].reshape(1, CS), lnp["b"].reshape(1, CS), vlng, e16, e8, e32)


def _t6_post(s, vx, vy, vz, rot9, trans, inv, noise, pr):
    B = 400
    grid = N // B

    def body(s_r, vx_r, vy_r, vz_r, r9_r, tr_r, inv_r, no_r,
             lc_w1, lc_b1, lc_ws, lc_bs, lc_wg, lc_bg, lc_wm,
             lr_w1, lr_b1, lr_ws, lr_bs, lr_wg, lr_bg, lr_wm,
             l3g, l3b, v3g,
             nt_w1, nt_b1, nt_w2, nt_b2, nt_g, nt_b, nt_wg, nt_bg, nt_wm,
             w6_r, b6_r, wv2_r,
             so_r, vxo_r, vyo_r, vzo_r, r9o_r, tro_r):
        s_ = s_r[...]
        v = [vx_r[...], vy_r[...], vz_r[...]]
        inv_b = inv_r[...]
        no_b = no_r[...]

        def gate(s_, v, w1, b1, ws, bs, wg, bg, wm, act):
            nrm = jnp.sqrt(v[0] ** 2 + v[1] ** 2 + v[2] ** 2 + 1e-8)
            h = jnp.maximum(_dot(jnp.concatenate([s_, nrm], axis=1), w1[...])
                            + b1[...], 0.0)
            su = _dot(h, ws[...]) + bs[...]
            g = act(_dot(h, wg[...]) + bg[...])
            vu = [g * _dot(v[i], wm[...]) for i in range(3)]
            return su, vu

        su, vu = gate(s_, v, lc_w1, lc_b1, lc_ws, lc_bs, lc_wg, lc_bg, lc_wm,
                      jax.nn.sigmoid)
        s_ = s_ + su * inv_b
        v = [v[i] + vu[i] * inv_b for i in range(3)]
        su, vu = gate(s_, v, lr_w1, lr_b1, lr_ws, lr_bs, lr_wg, lr_bg, lr_wm,
                      jnp.tanh)
        sn = s_ + su * inv_b

        def ln(x, g, b):
            mu = jnp.mean(x, axis=1, keepdims=True)
            var = jnp.mean((x - mu) ** 2, axis=1, keepdims=True)
            return (x - mu) / jnp.sqrt(var + 1e-5) * g[...] + b[...]

        s_ = ln(sn, l3g, l3b)
        v = [v[i] + vu[i] * inv_b for i in range(3)]
        n2 = jnp.mean(v[0] ** 2 + v[1] ** 2 + v[2] ** 2, axis=1, keepdims=True)
        v = [v[i] * (v3g[...] / jnp.sqrt(n2 + 1e-6)) for i in range(3)]

        h = jnp.maximum(_dot(s_, nt_w1[...]) + nt_b1[...], 0.0)
        s2 = ln(s_ + _dot(h, nt_w2[...]) + nt_b2[...], nt_g, nt_b)
        gg = jax.nn.sigmoid(_dot(s2, nt_wg[...]) + nt_bg[...])
        v = [v[i] + gg * _dot(v[i], nt_wm[...]) for i in range(3)]

        s_f = s2 * inv_b
        v_f = [v[i] * inv_b for i in range(3)]

        sb = s_f * no_b
        vb = [v_f[i] * no_b for i in range(3)]
        a = [_dot(vb[i], wv2_r[...]) for i in range(3)]
        vc = jnp.concatenate([a[0][:, :1], a[1][:, :1], a[2][:, :1],
                              a[0][:, 1:], a[1][:, 1:], a[2][:, 1:]], axis=1)
        upd = (_dot(sb, w6_r[...]) + b6_r[...] + vc) * no_b

        qn = jnp.sqrt(1.0 + upd[:, 0:1] ** 2 + upd[:, 1:2] ** 2
                      + upd[:, 2:3] ** 2)
        qw = 1.0 / qn
        qx = upd[:, 0:1] / qn
        qy = upd[:, 1:2] / qn
        qz = upd[:, 2:3] / qn
        ru = [1 - 2 * (qy * qy + qz * qz), 2 * (qx * qy - qz * qw),
              2 * (qx * qz + qy * qw),
              2 * (qx * qy + qz * qw), 1 - 2 * (qx * qx + qz * qz),
              2 * (qy * qz - qx * qw),
              2 * (qx * qz - qy * qw), 2 * (qy * qz + qx * qw),
              1 - 2 * (qx * qx + qy * qy)]
        r9 = r9_r[...]
        newr = [sum(r9[:, 3 * i + j:3 * i + j + 1] * ru[3 * j + k]
                    for j in range(3)) for i in range(3) for k in range(3)]
        tr = tr_r[...]
        newt = [tr[:, i:i + 1]
                + sum(r9[:, 3 * i + j:3 * i + j + 1] * upd[:, 3 + j:4 + j]
                      for j in range(3)) for i in range(3)]
        so_r[...] = s_f
        vxo_r[...] = v_f[0]
        vyo_r[...] = v_f[1]
        vzo_r[...] = v_f[2]
        r9o_r[...] = jnp.concatenate(newr, axis=1)
        tro_r[...] = jnp.concatenate(newt, axis=1)

    full = lambda shape: pl.BlockSpec(shape, lambda i: tuple(0 for _ in shape))
    row = lambda w: pl.BlockSpec((B, w), lambda i: (i, 0))
    lcu, lru, nt, bb = pr["lcu"], pr["lru"], pr["nt"], pr["bb"]
    return pl.pallas_call(
        body,
        grid=(grid,),
        in_specs=[row(CS), row(CV), row(CV), row(CV), row(9), row(3), row(1),
                  row(1),
                  full((CS + CV, CS)), full((1, CS)), full((CS, CS)),
                  full((1, CS)), full((CS, CV)), full((1, CV)), full((CV, CV)),
                  full((CS + CV, CS)), full((1, CS)), full((CS, CS)),
                  full((1, CS)), full((CS, CV)), full((1, CV)), full((CV, CV)),
                  full((1, CS)), full((1, CS)), full((1, CV)),
                  full((CS, 2 * CS)), full((1, 2 * CS)), full((2 * CS, CS)),
                  full((1, CS)), full((1, CS)), full((1, CS)),
                  full((CS, CV)), full((1, CV)), full((CV, CV)),
                  full((CS, 6)), full((1, 6)), full((CV, 2))],
        out_specs=[row(CS), row(CV), row(CV), row(CV), row(9), row(3)],
        out_shape=[jax.ShapeDtypeStruct((N, CS), jnp.float32),
                   jax.ShapeDtypeStruct((N, CV), jnp.float32),
                   jax.ShapeDtypeStruct((N, CV), jnp.float32),
                   jax.ShapeDtypeStruct((N, CV), jnp.float32),
                   jax.ShapeDtypeStruct((N, 9), jnp.float32),
                   jax.ShapeDtypeStruct((N, 3), jnp.float32)],
    )(s, vx, vy, vz, rot9, trans, inv, noise,
      lcu["w1"], lcu["b1"], lcu["ws"], lcu["bs"], lcu["wg"], lcu["bg"],
      lcu["wm"],
      lru["w1"], lru["b1"], lru["ws"], lru["bs"], lru["wg"], lru["bg"],
      lru["wm"],
      pr["ln_s3g"], pr["ln_s3b"], pr["ln_v3"],
      nt["w1"], nt["b1"], nt["w2"], nt["b2"], nt["g"], nt["b"], nt["wg"],
      nt["bg"], nt["wm"],
      bb["w6"], bb["b6"], bb["wv2"])


def _t7_edge_tr(sg, piece_src, piece_dst, z, p):
    B = 512
    grid = (E + B - 1) // B
    off_s = piece_src * (EP // B)
    off_d = piece_dst * (EP // B)

    def body(ss_r, sd_r, z_r, w1_r, b1_r, w2_r, b2_r, g_r, b_r, zo_r):
        z_ = z_r[...]
        hcat = jnp.concatenate([ss_r[...], sd_r[...], z_], axis=1)
        h = jnp.maximum(_dot(hcat, w1_r[...]) + b1_r[...], 0.0)
        zn = z_ + _dot(h, w2_r[...]) + b2_r[...]
        mu = jnp.mean(zn, axis=1, keepdims=True)
        var = jnp.mean((zn - mu) ** 2, axis=1, keepdims=True)
        zo_r[...] = (zn - mu) / jnp.sqrt(var + 1e-5) * g_r[...] + b_r[...]

    full = lambda shape: pl.BlockSpec(shape, lambda i: tuple(0 for _ in shape))
    return pl.pallas_call(
        body,
        grid=(grid,),
        in_specs=[pl.BlockSpec((B, CS), lambda i: (i + off_s, 0)),
                  pl.BlockSpec((B, CS), lambda i: (i + off_d, 0)),
                  pl.BlockSpec((B, CZ), lambda i: (i, 0)),
                  full((2 * CS + CZ, 2 * CZ)), full((1, 2 * CZ)),
                  full((2 * CZ, CZ)), full((1, CZ)),
                  full((1, CZ)), full((1, CZ))],
        out_specs=pl.BlockSpec((B, CZ), lambda i: (i, 0)),
        out_shape=jax.ShapeDtypeStruct((E, CZ), jnp.float32),
    )(sg, sg, z, p["w1"], p["b1"], p["w2"], p["b2"], p["g"], p["b"])


def _prep_psa(p):
    wo = p["wo"]
    o_part = wo[:CS]
    optl_part = wo[CS:CS + 192].reshape(64, 3, CS).transpose(1, 0, 2).reshape(192, CS)
    rest = wo[CS + 192:]
    return {
        "wq": p["wq"], "wk": p["wk"], "wv": p["wv"],
        "wqp3": p["wqp"].reshape(CS, H * PQ, 3).transpose(2, 0, 1),
        "wkp3": p["wkp"].reshape(CS, H * PQ, 3).transpose(2, 0, 1),
        "wvp3": p["wvp"].reshape(CS, H * PV, 3).transpose(2, 0, 1),
        "vq": p["vq"], "vk": p["vk"], "vv": p["vv"],
        "wb": p["wb"],
        "gsp": jax.nn.softplus(p["gamma"]).reshape(1, H),
        "woP": jnp.concatenate([o_part, optl_part, rest], axis=0),
        "bo": p["bo"].reshape(1, CS),
        "wpv": p["wpv"],
    }


def _pad_idx(idx, pad_val, rows):
    return jnp.pad(idx, (0, rows - idx.shape[0]), constant_values=pad_val)


def _attention(s, vx, vy, vz, rot9, trans, inv, z, ei, pp, lnp, vlng, consts):
    e16, e8, e32, selqk, seld2, zeros_blk = consts
    src = _pad_idx(ei[0], N, EP)
    dst = _pad_idx(ei[1], N, EP)
    src2d = src.reshape(EP // GCH, GCH)
    dst2d = dst.reshape(EP // GCH, GCH)
    dst2d_s = dst.reshape(EP // SCH, SCH)
    zp = jnp.pad(z, ((0, EP - E), (0, 0)))

    td, tsa, tsb = _t1_tables(s, vx, vy, vz, rot9, trans, pp)
    td_g = _sc_gather(td, dst2d, 256, EP)
    tsa_g = _sc_gather(tsa, src2d, 256, EP)
    tsb_g = _sc_gather(tsb, src2d, 384, EP)
    logits, midx = _t2_logits(td_g, tsa_g, zp, dst.reshape(EP, 1), pp,
                              selqk, seld2)
    midx_f = midx.reshape(EP * 16)
    parts = _sc_segmax(logits.reshape(EP * H), midx_f)
    m2 = _t3_mmerge(parts.reshape(NWORK, NT, 8))
    m_g = _sc_mgather(m2.reshape(NT * 8), midx_f).reshape(EP, 8)
    wps = _t4_weights(logits, m_g, tsb_g, zp, e16, e8, e32)
    acc = _sc_scatter(wps, dst2d_s, zeros_blk)
    return _t5_finalize(acc.reshape(2, NPASS, NT, CCOL), s, vx, vy, vz, rot9,
                        trans, inv, pp, lnp, vlng, e16, e8, e32)


def kernel(node_features, rigids_rot, rigids_trans, edge_features, edge_index,
           seq_edge_features, seq_edge_index, x_mask, noising_mask,
           node_vectors, params):
    f32 = jnp.float32
    s0 = node_features
    rot9 = rigids_rot.reshape(N, 9)
    trans = rigids_trans
    inv = (~x_mask).astype(f32).reshape(N, 1)
    noise = noising_mask.reshape(N, 1)
    v_pl = jnp.transpose(node_vectors, (2, 0, 1))
    vx, vy, vz = v_pl[0], v_pl[1], v_pl[2]

    hh = jnp.arange(H)
    e16 = (jnp.arange(128)[None, :] // 16 == hh[:, None]).astype(f32)
    e8 = (jnp.arange(64)[None, :] // 8 == hh[:, None]).astype(f32)
    e32 = (jnp.arange(256)[None, :] // 32 == hh[:, None]).astype(f32)
    selqk = e16.T
    seld2 = ((jnp.arange(96)[:, None] % 32) // 4 == hh[None, :]).astype(f32)
    zeros_blk = jnp.zeros((640, CCOL), f32)
    consts = (e16, e8, e32, selqk, seld2, zeros_blk)

    pA = _prep_psa(params["attn_seq"])
    pB = _prep_psa(params["attn_spatial"])

    s1, vx1, vy1, vz1 = _attention(
        s0, vx, vy, vz, rot9, trans, inv, seq_edge_features, seq_edge_index,
        pA, params["ln_s1"], params["ln_v1"].reshape(1, CV), consts)
    s2, vx2, vy2, vz2 = _attention(
        s1, vx1, vy1, vz1, rot9, trans, inv, edge_features, edge_index,
        pB, params["ln_s2"], params["ln_v2"].reshape(1, CV), consts)

    pr = {
        "lcu": {k: (v.reshape(1, -1) if v.ndim == 1 else v)
                for k, v in params["lcu"].items()},
        "lru": {k: (v.reshape(1, -1) if v.ndim == 1 else v)
                for k, v in params["lru"].items()},
        "nt": {k: (v.reshape(1, -1) if v.ndim == 1 else v)
               for k, v in params["nt"].items()},
        "bb": {"w6": params["bb"]["w6"], "b6": params["bb"]["b6"].reshape(1, 6),
               "wv2": params["bb"]["wv2"]},
        "ln_s3g": params["ln_s3"]["g"].reshape(1, CS),
        "ln_s3b": params["ln_s3"]["b"].reshape(1, CS),
        "ln_v3": params["ln_v3"].reshape(1, CV),
    }
    s3, vfx, vfy, vfz, r9n, trn = _t6_post(s2, vx2, vy2, vz2, rot9, trans,
                                           inv, noise, pr)

    srcB = _pad_idx(edge_index[0], 0, EP)
    dstB = _pad_idx(edge_index[1], 0, EP)
    srcA = _pad_idx(seq_edge_index[0], 0, EP)
    dstA = _pad_idx(seq_edge_index[1], 0, EP)
    idx_et = jnp.concatenate([srcB, dstB, srcA, dstA]).reshape(
        4 * EP // GCH, GCH)
    sg = _sc_gather(s3, idx_et, CS, 4 * EP)

    et = {k: (v.reshape(1, -1) if v.ndim == 1 else v)
          for k, v in params["et"].items()}
    set_ = {k: (v.reshape(1, -1) if v.ndim == 1 else v)
            for k, v in params["set"].items()}
    z_out = _t7_edge_tr(sg, 0, 1, edge_features, et)
    zs_out = _t7_edge_tr(sg, 2, 3, seq_edge_features, set_)

    v_final = jnp.stack([vfx, vfy, vfz], axis=-1)
    return (s3, r9n.reshape(N, 3, 3), trn, z_out, zs_out, v_final)


# speedup vs baseline: 16.0487x; 1.0400x over previous
import functools

import jax
import jax.numpy as jnp
from jax import lax
from jax.experimental import pallas as pl
from jax.experimental.pallas import tpu as pltpu
from jax.experimental.pallas import tpu_sc as plsc

N = 10000
E = 160000
CS = 128
CV = 8
CZ = 32
CH = 16
H = 8
PQ = 4
PV = 8

NT = 10240
EP = 163840
NEG = -3e38
HP = lax.Precision.HIGHEST

NWORK = 32
PER_W = EP // NWORK
GCH = 128
SCH = 128
CCOL = 128
NPASS = 5
ROW_W = 640

@functools.cache
def _mesh():
    return plsc.VectorSubcoreMesh(core_axis_name="c", subcore_axis_name="s")


def _wid():
    return lax.axis_index("s") * 2 + lax.axis_index("c")


def _sc_gather(table, idx, D, rows):
    gch = GCH if D <= 256 else GCH // 2
    idx2d = idx.reshape(rows // gch, gch)
    per_w = rows // NWORK
    chunks = per_w // gch

    @functools.partial(
        pl.kernel,
        out_type=jax.ShapeDtypeStruct((rows, D), jnp.float32),
        mesh=_mesh(),
        scratch_types=[
            pltpu.VMEM((chunks, gch), jnp.int32),
            pltpu.VMEM((2, gch, D), jnp.float32),
            pltpu.SemaphoreType.DMA,
            pltpu.SemaphoreType.DMA,
        ],
    )
    def k(table_hbm, idx_hbm, out_hbm, idx_v, rows_v, sem0, sem1):
        w = _wid()
        row0 = w * per_w
        chunk0 = w * chunks
        sems = (sem0, sem1)
        pltpu.sync_copy(idx_hbm.at[pl.ds(chunk0, chunks)], idx_v)
        for b in range(2):
            pltpu.async_copy(table_hbm.at[idx_v.at[b]], rows_v.at[b], sems[b])

        def pair(g, _):
            for b in range(2):
                c = g * 2 + b
                pltpu.make_async_copy(table_hbm.at[idx_v.at[c]],
                                      rows_v.at[b], sems[b]).wait()
                pltpu.sync_copy(rows_v.at[b],
                                out_hbm.at[pl.ds(row0 + c * gch, gch)])
                cn = jnp.minimum(c + 2, chunks - 1)
                pltpu.async_copy(table_hbm.at[idx_v.at[cn]], rows_v.at[b],
                                 sems[b])
            return 0

        lax.fori_loop(0, chunks // 2, pair, 0)
        for b in range(2):
            pltpu.make_async_copy(table_hbm.at[idx_v.at[chunks - 1]],
                                  rows_v.at[b], sems[b]).wait()

    return k(table, idx2d)


def _sc_segmax(lflat, midx):
    CH_E = 512
    chunks = PER_W // CH_E

    @functools.partial(
        pl.kernel,
        out_type=jax.ShapeDtypeStruct((NWORK, NT * 8), jnp.float32),
        mesh=_mesh(),
        compiler_params=pltpu.CompilerParams(needs_layout_passes=False),
        scratch_types=[
            pltpu.VMEM((NT * 8,), jnp.float32),
            pltpu.VMEM((CH_E * 8,), jnp.float32),
            pltpu.VMEM((CH_E * 16,), jnp.int32),
        ],
    )
    def k(l_hbm, mi_hbm, out_hbm, m_v, l_v, i_v):
        w = _wid()
        base = w * PER_W
        negv = jnp.full((16,), NEG, jnp.float32)
        lanes = lax.iota(jnp.int32, 16)
        lo_mask = lanes < 8

        def init(i, _):
            m_v[pl.ds(i * 16, 16)] = negv
            return 0

        lax.fori_loop(0, NT * 8 // 16, init, 0)

        def chunk(c, _):
            pltpu.sync_copy(l_hbm.at[pl.ds((base + c * CH_E) * 8, CH_E * 8)], l_v)
            pltpu.sync_copy(mi_hbm.at[pl.ds((base + c * CH_E) * 16, CH_E * 16)],
                            i_v)

            def edge(e, _):
                iv = i_v[pl.ds(e * 16, 16)]
                lv = l_v[pl.ds(e * 8, 16)]
                lsel = jnp.where(lo_mask, lv, negv)
                mv = plsc.load_gather(m_v, [iv])
                plsc.store_scatter(m_v, [iv], jnp.maximum(mv, lsel), mask=lo_mask)
                return 0

            lax.fori_loop(0, CH_E, edge, 0)
            return 0

        lax.fori_loop(0, chunks, chunk, 0)
        pltpu.sync_copy(m_v, out_hbm.at[w])

    return k(lflat, midx)


def _sc_scatter(wps, idx2d, zeros_blk):
    chunks = PER_W // SCH

    @functools.partial(
        pl.kernel,
        out_type=jax.ShapeDtypeStruct((2 * NPASS * NT, CCOL), jnp.float32),
        mesh=_mesh(),
        scratch_types=[
            pltpu.VMEM_SHARED((NT, CCOL), jnp.float32),
            pltpu.VMEM((chunks, SCH), jnp.int32),
            pltpu.VMEM((2, SCH, CCOL), jnp.float32),
            pltpu.SemaphoreType.DMA,
            pltpu.SemaphoreType.DMA,
        ],
    )
    def k(w0_h, w1_h, w2_h, w3_h, w4_h, idx_h, z_h, out_h, acc, idx_v, v_buf,
          sem0, sem1):
        cid = lax.axis_index("c")
        sid = lax.axis_index("s")
        w = sid * 2 + cid
        row0 = w * PER_W
        sems = (sem0, sem1)
        pltpu.sync_copy(idx_h.at[pl.ds(w * chunks, chunks)], idx_v)
        for p, wp in enumerate((w0_h, w1_h, w2_h, w3_h, w4_h)):
            pltpu.sync_copy(z_h, acc.at[pl.ds(sid * 640, 640)])
            plsc.subcore_barrier()
            for b in range(2):
                pltpu.async_copy(wp.at[pl.ds(row0 + b * SCH, SCH)],
                                 v_buf.at[b], sems[b])

            def pair(g, _):
                for b in range(2):
                    c = g * 2 + b
                    pltpu.make_async_copy(wp.at[pl.ds(row0 + c * SCH, SCH)],
                                          v_buf.at[b], sems[b]).wait()
                    pltpu.sync_copy(v_buf.at[b], acc.at[idx_v.at[c]], add=True)
                    cn = jnp.minimum(c + 2, chunks - 1)
                    pltpu.async_copy(wp.at[pl.ds(row0 + cn * SCH, SCH)],
                                     v_buf.at[b], sems[b])
                return 0

            lax.fori_loop(0, chunks // 2, pair, 0)
            for b in range(2):
                pltpu.make_async_copy(
                    wp.at[pl.ds(row0 + (chunks - 1) * SCH, SCH)],
                    v_buf.at[b], sems[b]).wait()
            plsc.subcore_barrier()
            slot = cid * NPASS + p
            pltpu.sync_copy(acc.at[pl.ds(sid * 640, 640)],
                            out_h.at[pl.ds(slot * NT + sid * 640, 640)])
            plsc.subcore_barrier()

    return k(*wps, idx2d, zeros_blk)


def _sc_mgather(m2flat, midx):
    CH_E = 512
    chunks = PER_W // CH_E

    @functools.partial(
        pl.kernel,
        out_type=jax.ShapeDtypeStruct((EP * 8,), jnp.float32),
        mesh=_mesh(),
        compiler_params=pltpu.CompilerParams(needs_layout_passes=False),
        scratch_types=[
            pltpu.VMEM((NT * 8,), jnp.float32),
            pltpu.VMEM((CH_E * 16,), jnp.int32),
            pltpu.VMEM((CH_E * 8 + 16,), jnp.float32),
        ],
    )
    def k(m_hbm, mi_hbm, out_hbm, m_v, i_v, o_v):
        w = _wid()
        base = w * PER_W
        lanes = lax.iota(jnp.int32, 16)
        lo_mask = lanes < 8

        def tload(t, _):
            pltpu.sync_copy(m_hbm.at[pl.ds(t * 8192, 8192)],
                            m_v.at[pl.ds(t * 8192, 8192)])
            return 0

        lax.fori_loop(0, NT * 8 // 8192, tload, 0)

        def chunk(c, _):
            pltpu.sync_copy(mi_hbm.at[pl.ds((base + c * CH_E) * 16, CH_E * 16)],
                            i_v)

            def edge(e, _):
                iv = i_v[pl.ds(e * 16, 16)]
                mv = plsc.load_gather(m_v, [jnp.where(lo_mask, iv, 0)])
                o_v[pl.ds(e * 8, 16)] = mv
                return 0

            lax.fori_loop(0, CH_E, edge, 0)
            pltpu.sync_copy(o_v.at[pl.ds(0, CH_E * 8)],
                            out_hbm.at[pl.ds((base + c * CH_E) * 8, CH_E * 8)])
            return 0

        lax.fori_loop(0, chunks, chunk, 0)

    return k(m2flat, midx)


def _dot(a, b):
    return jnp.dot(a, b, precision=HP)


def _t1_tables(s, vx, vy, vz, rot9, trans, p):
    B = 512
    grid = NT // B

    def body(s_r, vx_r, vy_r, vz_r, r9_r, tr_r, wq_r, wk_r, wv_r, wqp_r, wkp_r,
             wvp_r, vq_r, vk_r, vv_r, td_r, tsab_r):
        sb = s_r[...]
        vpl = (vx_r[...], vy_r[...], vz_r[...])
        r9 = r9_r[...]
        tr = tr_r[...]
        q = _dot(sb, wq_r[...])
        kk = _dot(sb, wk_r[...])
        vs = _dot(sb, wv_r[...])
        qp = [_dot(sb, wqp_r[j]) + _dot(vpl[j], vq_r[...]) for j in range(3)]
        kp = [_dot(sb, wkp_r[j]) + _dot(vpl[j], vk_r[...]) for j in range(3)]
        vp = [_dot(sb, wvp_r[j]) + _dot(vpl[j], vv_r[...]) for j in range(3)]

        def glob(pts, i):
            return (r9[:, 3 * i:3 * i + 1] * pts[0]
                    + r9[:, 3 * i + 1:3 * i + 2] * pts[1]
                    + r9[:, 3 * i + 2:3 * i + 3] * pts[2]
                    + tr[:, i:i + 1])

        qpg = [glob(qp, i) for i in range(3)]
        kpg = [glob(kp, i) for i in range(3)]
        vpg = [glob(vp, i) for i in range(3)]
        rid = pl.program_id(0) * B + lax.broadcasted_iota(jnp.int32, (B, 1), 0)
        valid = rid < N
        z32 = jnp.zeros((B, 32), jnp.float32)
        td = jnp.where(valid, jnp.concatenate([q] + qpg + [z32], axis=1), 0.0)
        tsab = jnp.where(
            valid,
            jnp.concatenate([kk] + kpg + [z32, vs] + vpg + [z32, z32], axis=1),
            0.0)
        td_r[...] = td
        tsab_r[...] = tsab

    full = lambda shape: pl.BlockSpec(shape, lambda i: tuple(0 for _ in shape))
    row = lambda w: pl.BlockSpec((B, w), lambda i: (i, 0))
    return pl.pallas_call(
        body,
        grid=(grid,),
        in_specs=[row(CS), row(CV), row(CV), row(CV), row(9), row(3),
                  full((CS, CS)), full((CS, CS)), full((CS, CS)),
                  full((3, CS, H * PQ)), full((3, CS, H * PQ)),
                  full((3, CS, H * PV)),
                  full((CV, H * PQ)), full((CV, H * PQ)), full((CV, H * PV))],
        out_specs=[row(256), row(640)],
        out_shape=[jax.ShapeDtypeStruct((NT, 256), jnp.float32),
                   jax.ShapeDtypeStruct((NT, 640), jnp.float32)],
    )(s, vx, vy, vz, rot9, trans, p["wq"], p["wk"], p["wv"], p["wqp3"],
      p["wkp3"], p["wvp3"], p["vq"], p["vk"], p["vv"])


def _t2_logits(td_g, tsa_g, zp, dstcol, p, selqk, seld2):
    B = 512
    grid = EP // B

    def body(td_r, tsa_r, z_r, d_r, wb_r, g_r, sq_r, sd_r, l_r, mi_r):
        td = td_r[...]
        tsa = tsa_r[...]
        z = z_r[...]
        lq = _dot(td[:, :CS] * tsa[:, :CS], sq_r[...]) * 0.25
        d2 = _dot((td[:, CS:224] - tsa[:, CS:224]) ** 2, sd_r[...])
        logits = lq + _dot(z, wb_r[...]) - 0.5 * g_r[...] * d2
        eid = pl.program_id(0) * B + lax.broadcasted_iota(jnp.int32, (B, H), 0)
        l_r[...] = jnp.where(eid < E, logits, NEG)
        mi_r[...] = d_r[...] * 8 + lax.broadcasted_iota(jnp.int32, (B, 16), 1)

    full = lambda shape: pl.BlockSpec(shape, lambda i: tuple(0 for _ in shape))
    row = lambda w: pl.BlockSpec((B, w), lambda i: (i, 0))
    return pl.pallas_call(
        body,
        grid=(grid,),
        in_specs=[row(256), row(256), row(CZ), row(1),
                  full((CZ, H)), full((1, H)), full((CS, H)), full((96, H))],
        out_specs=[row(H), row(16)],
        out_shape=[jax.ShapeDtypeStruct((EP, H), jnp.float32),
                   jax.ShapeDtypeStruct((EP, 16), jnp.int32)],
    )(td_g, tsa_g, zp, dstcol, p["wb"], p["gsp"], selqk, seld2)


def _t3_mmerge(parts):
    B = 512
    grid = NT // B

    def body(p_r, m_r):
        x = p_r[...]
        m = x[0]
        for i in range(1, NWORK):
            m = jnp.maximum(m, x[i])
        m_r[...] = m

    return pl.pallas_call(
        body,
        grid=(grid,),
        in_specs=[pl.BlockSpec((NWORK, B, 8), lambda i: (0, i, 0))],
        out_specs=pl.BlockSpec((B, 8), lambda i: (i, 0)),
        out_shape=jax.ShapeDtypeStruct((NT, 8), jnp.float32),
    )(parts)


def _t4_weights(logits, m_g, tsb_g, zp, e16, e8, e32):
    B = 512
    grid = EP // B

    def body(l_r, m_r, tsb_r, z_r, e16_r, e8_r, e32_r, w0_r, w1_r, w2_r, w3_r,
             w4_r):
        w = jnp.exp(l_r[...] - m_r[...])
        tsb = tsb_r[...][:, 256:]
        z = z_r[...]
        r16 = _dot(w, e16_r[...])
        r8 = _dot(w, e8_r[...])
        r32 = _dot(w, e32_r[...])
        ztile = jnp.concatenate([z] * H, axis=1)
        cat = jnp.concatenate(
            [r16 * tsb[:, :CS]]
            + [r8 * tsb[:, CS + 64 * j:CS + 64 * (j + 1)] for j in range(3)]
            + [r32 * ztile, w, jnp.zeros((B, 56), jnp.float32)], axis=1)
        for i, o_r in enumerate((w0_r, w1_r, w2_r, w3_r, w4_r)):
            o_r[...] = cat[:, i * CCOL:(i + 1) * CCOL]

    full = lambda shape: pl.BlockSpec(shape, lambda i: tuple(0 for _ in shape))
    row = lambda w: pl.BlockSpec((B, w), lambda i: (i, 0))
    return pl.pallas_call(
        body,
        grid=(grid,),
        in_specs=[row(H), row(H), row(640), row(CZ),
                  full((H, 128)), full((H, 64)), full((H, 256))],
        out_specs=[row(CCOL)] * NPASS,
        out_shape=[jax.ShapeDtypeStruct((EP, CCOL), jnp.float32)] * NPASS,
    )(logits, m_g, tsb_g, zp, e16, e8, e32)


def _t5_finalize(acc, s, vx, vy, vz, rot9, trans, inv, p, lnp, vlng,
                 e16, e8, e32):
    B = 400
    grid = N // B

    def body(a_r, s_r, vx_r, vy_r, vz_r, r9_r, tr_r, inv_r, wo_r, bo_r, wpv_r,
             g_r, b_r, vg_r, e16_r, e8_r, e32_r, so_r, vxo_r, vyo_r, vzo_r):
        a = a_r[...]
        acc2 = a[0] + a[1]
        flat = jnp.concatenate([acc2[j] for j in range(NPASS)], axis=1)
        den = flat[:, 576:584] + 1e-9
        d16 = _dot(den, e16_r[...])
        d8 = _dot(den, e8_r[...])
        d32 = _dot(den, e32_r[...])
        o = flat[:, :CS] / d16
        r9 = r9_r[...]
        tr = tr_r[...]
        opt = [flat[:, CS + 64 * j:CS + 64 * (j + 1)] / d8 for j in range(3)]
        optl = [sum((r9[:, 3 * j + i:3 * j + i + 1]
                     * (opt[j] - tr[:, j:j + 1])) for j in range(3))
                for i in range(3)]
        onorm = jnp.sqrt(optl[0] ** 2 + optl[1] ** 2 + optl[2] ** 2 + 1e-8)
        ozn = flat[:, 320:576] / d32
        feats = jnp.concatenate([o] + optl + [onorm, ozn], axis=1)
        su = _dot(feats, wo_r[...]) + bo_r[...]
        inv_b = inv_r[...]
        sn = s_r[...] + su * inv_b
        mu = jnp.mean(sn, axis=1, keepdims=True)
        var = jnp.mean((sn - mu) ** 2, axis=1, keepdims=True)
        so_r[...] = (sn - mu) / jnp.sqrt(var + 1e-5) * g_r[...] + b_r[...]
        vn = [v_r[...] + _dot(optl[i], wpv_r[...]) * inv_b
              for i, v_r in enumerate((vx_r, vy_r, vz_r))]
        n2 = jnp.mean(vn[0] ** 2 + vn[1] ** 2 + vn[2] ** 2, axis=1,
                      keepdims=True)
        scale = vg_r[...] / jnp.sqrt(n2 + 1e-6)
        vxo_r[...] = vn[0] * scale
        vyo_r[...] = vn[1] * scale
        vzo_r[...] = vn[2] * scale

    full = lambda shape: pl.BlockSpec(shape, lambda i: tuple(0 for _ in shape))
    row = lambda w: pl.BlockSpec((B, w), lambda i: (i, 0))
    return pl.pallas_call(
        body,
        grid=(grid,),
        in_specs=[pl.BlockSpec((2, NPASS, B, CCOL), lambda i: (0, 0, i, 0)),
                  row(CS), row(CV), row(CV), row(CV), row(9), row(3), row(1),
                  full((640, CS)), full((1, CS)), full((64, CV)),
                  full((1, CS)), full((1, CS)), full((1, CV)),
                  full((H, 128)), full((H, 64)), full((H, 256))],
        out_specs=[row(CS), row(CV), row(CV), row(CV)],
        out_shape=[jax.ShapeDtypeStruct((N, CS), jnp.float32)]
        + [jax.ShapeDtypeStruct((N, CV), jnp.float32)] * 3,
    )(acc, s, vx, vy, vz, rot9, trans, inv, p["woP"], p["bo"], p["wpv"],
      lnp["g"].reshape(1, CS), lnp["b"].reshape(1, CS), vlng, e16, e8, e32)


def _t6_post(s, vx, vy, vz, rot9, trans, inv, noise, pr):
    B = 400
    grid = N // B

    def body(s_r, vx_r, vy_r, vz_r, r9_r, tr_r, inv_r, no_r,
             lc_w1, lc_b1, lc_ws, lc_bs, lc_wg, lc_bg, lc_wm,
             lr_w1, lr_b1, lr_ws, lr_bs, lr_wg, lr_bg, lr_wm,
             l3g, l3b, v3g,
             nt_w1, nt_b1, nt_w2, nt_b2, nt_g, nt_b, nt_wg, nt_bg, nt_wm,
             w6_r, b6_r, wv2_r,
             so_r, vxo_r, vyo_r, vzo_r, r9o_r, tro_r):
        s_ = s_r[...]
        v = [vx_r[...], vy_r[...], vz_r[...]]
        inv_b = inv_r[...]
        no_b = no_r[...]

        def gate(s_, v, w1, b1, ws, bs, wg, bg, wm, act):
            nrm = jnp.sqrt(v[0] ** 2 + v[1] ** 2 + v[2] ** 2 + 1e-8)
            h = jnp.maximum(_dot(jnp.concatenate([s_, nrm], axis=1), w1[...])
                            + b1[...], 0.0)
            su = _dot(h, ws[...]) + bs[...]
            g = act(_dot(h, wg[...]) + bg[...])
            vu = [g * _dot(v[i], wm[...]) for i in range(3)]
            return su, vu

        su, vu = gate(s_, v, lc_w1, lc_b1, lc_ws, lc_bs, lc_wg, lc_bg, lc_wm,
                      jax.nn.sigmoid)
        s_ = s_ + su * inv_b
        v = [v[i] + vu[i] * inv_b for i in range(3)]
        su, vu = gate(s_, v, lr_w1, lr_b1, lr_ws, lr_bs, lr_wg, lr_bg, lr_wm,
                      jnp.tanh)
        sn = s_ + su * inv_b

        def ln(x, g, b):
            mu = jnp.mean(x, axis=1, keepdims=True)
            var = jnp.mean((x - mu) ** 2, axis=1, keepdims=True)
            return (x - mu) / jnp.sqrt(var + 1e-5) * g[...] + b[...]

        s_ = ln(sn, l3g, l3b)
        v = [v[i] + vu[i] * inv_b for i in range(3)]
        n2 = jnp.mean(v[0] ** 2 + v[1] ** 2 + v[2] ** 2, axis=1, keepdims=True)
        v = [v[i] * (v3g[...] / jnp.sqrt(n2 + 1e-6)) for i in range(3)]

        h = jnp.maximum(_dot(s_, nt_w1[...]) + nt_b1[...], 0.0)
        s2 = ln(s_ + _dot(h, nt_w2[...]) + nt_b2[...], nt_g, nt_b)
        gg = jax.nn.sigmoid(_dot(s2, nt_wg[...]) + nt_bg[...])
        v = [v[i] + gg * _dot(v[i], nt_wm[...]) for i in range(3)]

        s_f = s2 * inv_b
        v_f = [v[i] * inv_b for i in range(3)]

        sb = s_f * no_b
        vb = [v_f[i] * no_b for i in range(3)]
        a = [_dot(vb[i], wv2_r[...]) for i in range(3)]
        vc = jnp.concatenate([a[0][:, :1], a[1][:, :1], a[2][:, :1],
                              a[0][:, 1:], a[1][:, 1:], a[2][:, 1:]], axis=1)
        upd = (_dot(sb, w6_r[...]) + b6_r[...] + vc) * no_b

        qn = jnp.sqrt(1.0 + upd[:, 0:1] ** 2 + upd[:, 1:2] ** 2
                      + upd[:, 2:3] ** 2)
        qw = 1.0 / qn
        qx = upd[:, 0:1] / qn
        qy = upd[:, 1:2] / qn
        qz = upd[:, 2:3] / qn
        ru = [1 - 2 * (qy * qy + qz * qz), 2 * (qx * qy - qz * qw),
              2 * (qx * qz + qy * qw),
              2 * (qx * qy + qz * qw), 1 - 2 * (qx * qx + qz * qz),
              2 * (qy * qz - qx * qw),
              2 * (qx * qz - qy * qw), 2 * (qy * qz + qx * qw),
              1 - 2 * (qx * qx + qy * qy)]
        r9 = r9_r[...]
        newr = [sum(r9[:, 3 * i + j:3 * i + j + 1] * ru[3 * j + k]
                    for j in range(3)) for i in range(3) for k in range(3)]
        tr = tr_r[...]
        newt = [tr[:, i:i + 1]
                + sum(r9[:, 3 * i + j:3 * i + j + 1] * upd[:, 3 + j:4 + j]
                      for j in range(3)) for i in range(3)]
        so_r[...] = s_f
        vxo_r[...] = v_f[0]
        vyo_r[...] = v_f[1]
        vzo_r[...] = v_f[2]
        r9o_r[...] = jnp.concatenate(newr, axis=1)
        tro_r[...] = jnp.concatenate(newt, axis=1)

    full = lambda shape: pl.BlockSpec(shape, lambda i: tuple(0 for _ in shape))
    row = lambda w: pl.BlockSpec((B, w), lambda i: (i, 0))
    lcu, lru, nt, bb = pr["lcu"], pr["lru"], pr["nt"], pr["bb"]
    return pl.pallas_call(
        body,
        grid=(grid,),
        in_specs=[row(CS), row(CV), row(CV), row(CV), row(9), row(3), row(1),
                  row(1),
                  full((CS + CV, CS)), full((1, CS)), full((CS, CS)),
                  full((1, CS)), full((CS, CV)), full((1, CV)), full((CV, CV)),
                  full((CS + CV, CS)), full((1, CS)), full((CS, CS)),
                  full((1, CS)), full((CS, CV)), full((1, CV)), full((CV, CV)),
                  full((1, CS)), full((1, CS)), full((1, CV)),
                  full((CS, 2 * CS)), full((1, 2 * CS)), full((2 * CS, CS)),
                  full((1, CS)), full((1, CS)), full((1, CS)),
                  full((CS, CV)), full((1, CV)), full((CV, CV)),
                  full((CS, 6)), full((1, 6)), full((CV, 2))],
        out_specs=[row(CS), row(CV), row(CV), row(CV), row(9), row(3)],
        out_shape=[jax.ShapeDtypeStruct((N, CS), jnp.float32),
                   jax.ShapeDtypeStruct((N, CV), jnp.float32),
                   jax.ShapeDtypeStruct((N, CV), jnp.float32),
                   jax.ShapeDtypeStruct((N, CV), jnp.float32),
                   jax.ShapeDtypeStruct((N, 9), jnp.float32),
                   jax.ShapeDtypeStruct((N, 3), jnp.float32)],
    )(s, vx, vy, vz, rot9, trans, inv, noise,
      lcu["w1"], lcu["b1"], lcu["ws"], lcu["bs"], lcu["wg"], lcu["bg"],
      lcu["wm"],
      lru["w1"], lru["b1"], lru["ws"], lru["bs"], lru["wg"], lru["bg"],
      lru["wm"],
      pr["ln_s3g"], pr["ln_s3b"], pr["ln_v3"],
      nt["w1"], nt["b1"], nt["w2"], nt["b2"], nt["g"], nt["b"], nt["wg"],
      nt["bg"], nt["wm"],
      bb["w6"], bb["b6"], bb["wv2"])


def _t7_edge_tr(sg, piece_src, piece_dst, z, p):
    B = 512
    grid = (E + B - 1) // B
    off_s = piece_src * (EP // B)
    off_d = piece_dst * (EP // B)

    def body(ss_r, sd_r, z_r, w1_r, b1_r, w2_r, b2_r, g_r, b_r, zo_r):
        z_ = z_r[...]
        hcat = jnp.concatenate([ss_r[...], sd_r[...], z_], axis=1)
        h = jnp.maximum(_dot(hcat, w1_r[...]) + b1_r[...], 0.0)
        zn = z_ + _dot(h, w2_r[...]) + b2_r[...]
        mu = jnp.mean(zn, axis=1, keepdims=True)
        var = jnp.mean((zn - mu) ** 2, axis=1, keepdims=True)
        zo_r[...] = (zn - mu) / jnp.sqrt(var + 1e-5) * g_r[...] + b_r[...]

    full = lambda shape: pl.BlockSpec(shape, lambda i: tuple(0 for _ in shape))
    return pl.pallas_call(
        body,
        grid=(grid,),
        in_specs=[pl.BlockSpec((B, CS), lambda i: (i + off_s, 0)),
                  pl.BlockSpec((B, CS), lambda i: (i + off_d, 0)),
                  pl.BlockSpec((B, CZ), lambda i: (i, 0)),
                  full((2 * CS + CZ, 2 * CZ)), full((1, 2 * CZ)),
                  full((2 * CZ, CZ)), full((1, CZ)),
                  full((1, CZ)), full((1, CZ))],
        out_specs=pl.BlockSpec((B, CZ), lambda i: (i, 0)),
        out_shape=jax.ShapeDtypeStruct((E, CZ), jnp.float32),
    )(sg, sg, z, p["w1"], p["b1"], p["w2"], p["b2"], p["g"], p["b"])


def _prep_psa(p):
    wo = p["wo"]
    o_part = wo[:CS]
    optl_part = wo[CS:CS + 192].reshape(64, 3, CS).transpose(1, 0, 2).reshape(192, CS)
    rest = wo[CS + 192:]
    return {
        "wq": p["wq"], "wk": p["wk"], "wv": p["wv"],
        "wqp3": p["wqp"].reshape(CS, H * PQ, 3).transpose(2, 0, 1),
        "wkp3": p["wkp"].reshape(CS, H * PQ, 3).transpose(2, 0, 1),
        "wvp3": p["wvp"].reshape(CS, H * PV, 3).transpose(2, 0, 1),
        "vq": p["vq"], "vk": p["vk"], "vv": p["vv"],
        "wb": p["wb"],
        "gsp": jax.nn.softplus(p["gamma"]).reshape(1, H),
        "woP": jnp.concatenate([o_part, optl_part, rest], axis=0),
        "bo": p["bo"].reshape(1, CS),
        "wpv": p["wpv"],
    }


def _pad_idx(idx, pad_val, rows):
    return jnp.pad(idx, (0, rows - idx.shape[0]), constant_values=pad_val)


def _attention(s, vx, vy, vz, rot9, trans, inv, z, ei, pp, lnp, vlng, consts):
    e16, e8, e32, selqk, seld2, zeros_blk = consts
    src = _pad_idx(ei[0], N, EP)
    dst = _pad_idx(ei[1], N, EP)
    dst2d_s = dst.reshape(EP // SCH, SCH)
    zp = jnp.pad(z, ((0, EP - E), (0, 0)))

    td, tsab = _t1_tables(s, vx, vy, vz, rot9, trans, pp)
    td_g = _sc_gather(td, dst, 256, EP)
    tsab_g = _sc_gather(tsab, src, 640, EP)
    logits, midx = _t2_logits(td_g, tsab_g, zp, dst.reshape(EP, 1), pp,
                              selqk, seld2)
    midx_f = midx.reshape(EP * 16)
    parts = _sc_segmax(logits.reshape(EP * H), midx_f)
    m2 = _t3_mmerge(parts.reshape(NWORK, NT, 8))
    m_g = _sc_mgather(m2.reshape(NT * 8), midx_f).reshape(EP, 8)
    wps = _t4_weights(logits, m_g, tsab_g, zp, e16, e8, e32)
    acc = _sc_scatter(wps, dst2d_s, zeros_blk)
    return _t5_finalize(acc.reshape(2, NPASS, NT, CCOL), s, vx, vy, vz, rot9,
                        trans, inv, pp, lnp, vlng, e16, e8, e32)


def kernel(node_features, rigids_rot, rigids_trans, edge_features, edge_index,
           seq_edge_features, seq_edge_index, x_mask, noising_mask,
           node_vectors, params):
    f32 = jnp.float32
    s0 = node_features
    rot9 = rigids_rot.reshape(N, 9)
    trans = rigids_trans
    inv = (~x_mask).astype(f32).reshape(N, 1)
    noise = noising_mask.reshape(N, 1)
    v_pl = jnp.transpose(node_vectors, (2, 0, 1))
    vx, vy, vz = v_pl[0], v_pl[1], v_pl[2]

    hh = jnp.arange(H)
    e16 = (jnp.arange(128)[None, :] // 16 == hh[:, None]).astype(f32)
    e8 = (jnp.arange(64)[None, :] // 8 == hh[:, None]).astype(f32)
    e32 = (jnp.arange(256)[None, :] // 32 == hh[:, None]).astype(f32)
    selqk = e16.T
    seld2 = ((jnp.arange(96)[:, None] % 32) // 4 == hh[None, :]).astype(f32)
    zeros_blk = jnp.zeros((640, CCOL), f32)
    consts = (e16, e8, e32, selqk, seld2, zeros_blk)

    pA = _prep_psa(params["attn_seq"])
    pB = _prep_psa(params["attn_spatial"])

    s1, vx1, vy1, vz1 = _attention(
        s0, vx, vy, vz, rot9, trans, inv, seq_edge_features, seq_edge_index,
        pA, params["ln_s1"], params["ln_v1"].reshape(1, CV), consts)
    s2, vx2, vy2, vz2 = _attention(
        s1, vx1, vy1, vz1, rot9, trans, inv, edge_features, edge_index,
        pB, params["ln_s2"], params["ln_v2"].reshape(1, CV), consts)

    pr = {
        "lcu": {k: (v.reshape(1, -1) if v.ndim == 1 else v)
                for k, v in params["lcu"].items()},
        "lru": {k: (v.reshape(1, -1) if v.ndim == 1 else v)
                for k, v in params["lru"].items()},
        "nt": {k: (v.reshape(1, -1) if v.ndim == 1 else v)
               for k, v in params["nt"].items()},
        "bb": {"w6": params["bb"]["w6"], "b6": params["bb"]["b6"].reshape(1, 6),
               "wv2": params["bb"]["wv2"]},
        "ln_s3g": params["ln_s3"]["g"].reshape(1, CS),
        "ln_s3b": params["ln_s3"]["b"].reshape(1, CS),
        "ln_v3": params["ln_v3"].reshape(1, CV),
    }
    s3, vfx, vfy, vfz, r9n, trn = _t6_post(s2, vx2, vy2, vz2, rot9, trans,
                                           inv, noise, pr)

    srcB = _pad_idx(edge_index[0], 0, EP)
    dstB = _pad_idx(edge_index[1], 0, EP)
    srcA = _pad_idx(seq_edge_index[0], 0, EP)
    dstA = _pad_idx(seq_edge_index[1], 0, EP)
    idx_et = jnp.concatenate([srcB, dstB, srcA, dstA])
    sg = _sc_gather(s3, idx_et, CS, 4 * EP)

    et = {k: (v.reshape(1, -1) if v.ndim == 1 else v)
          for k, v in params["et"].items()}
    set_ = {k: (v.reshape(1, -1) if v.ndim == 1 else v)
            for k, v in params["set"].items()}
    z_out = _t7_edge_tr(sg, 0, 1, edge_features, et)
    zs_out = _t7_edge_tr(sg, 2, 3, seq_edge_features, set_)

    v_final = jnp.stack([vfx, vfy, vfz], axis=-1)
    return (s3, r9n.reshape(N, 3, 3), trn, z_out, zs_out, v_final)


# speedup vs baseline: 16.0741x; 1.0016x over previous
import functools

import jax
import jax.numpy as jnp
from jax import lax
from jax.experimental import pallas as pl
from jax.experimental.pallas import tpu as pltpu
from jax.experimental.pallas import tpu_sc as plsc

N = 10000
E = 160000
CS = 128
CV = 8
CZ = 32
CH = 16
H = 8
PQ = 4
PV = 8

NT = 10240
EP = 163840
NEG = -3e38
HP = lax.Precision.HIGHEST

NWORK = 32
PER_W = EP // NWORK
GCH = 128
SCH = 128
CCOL = 128
NPASS = 5
ROW_W = 640

@functools.cache
def _mesh():
    return plsc.VectorSubcoreMesh(core_axis_name="c", subcore_axis_name="s")


def _wid():
    return lax.axis_index("s") * 2 + lax.axis_index("c")


def _sc_gather(table, idx, D, rows):
    gch = GCH if D <= 256 else GCH // 2
    idx2d = idx.reshape(rows // gch, gch)
    per_w = rows // NWORK
    chunks = per_w // gch

    @functools.partial(
        pl.kernel,
        out_type=jax.ShapeDtypeStruct((rows, D), jnp.float32),
        mesh=_mesh(),
        scratch_types=[
            pltpu.VMEM((chunks, gch), jnp.int32),
            pltpu.VMEM((2, gch, D), jnp.float32),
            pltpu.SemaphoreType.DMA,
            pltpu.SemaphoreType.DMA,
        ],
    )
    def k(table_hbm, idx_hbm, out_hbm, idx_v, rows_v, sem0, sem1):
        w = _wid()
        row0 = w * per_w
        chunk0 = w * chunks
        sems = (sem0, sem1)
        pltpu.sync_copy(idx_hbm.at[pl.ds(chunk0, chunks)], idx_v)
        for b in range(2):
            pltpu.async_copy(table_hbm.at[idx_v.at[b]], rows_v.at[b], sems[b])

        def pair(g, _):
            for b in range(2):
                c = g * 2 + b
                pltpu.make_async_copy(table_hbm.at[idx_v.at[c]],
                                      rows_v.at[b], sems[b]).wait()
                pltpu.sync_copy(rows_v.at[b],
                                out_hbm.at[pl.ds(row0 + c * gch, gch)])
                cn = jnp.minimum(c + 2, chunks - 1)
                pltpu.async_copy(table_hbm.at[idx_v.at[cn]], rows_v.at[b],
                                 sems[b])
            return 0

        lax.fori_loop(0, chunks // 2, pair, 0)
        for b in range(2):
            pltpu.make_async_copy(table_hbm.at[idx_v.at[chunks - 1]],
                                  rows_v.at[b], sems[b]).wait()

    return k(table, idx2d)


def _sc_segmax(lflat, midx):
    CH_E = 512
    chunks = PER_W // CH_E

    @functools.partial(
        pl.kernel,
        out_type=jax.ShapeDtypeStruct((NWORK, NT * 8), jnp.float32),
        mesh=_mesh(),
        compiler_params=pltpu.CompilerParams(needs_layout_passes=False),
        scratch_types=[
            pltpu.VMEM((NT * 8,), jnp.float32),
            pltpu.VMEM((2, CH_E * 8 + 16), jnp.float32),
            pltpu.VMEM((2, CH_E * 16), jnp.int32),
            pltpu.SemaphoreType.DMA,
            pltpu.SemaphoreType.DMA,
        ],
    )
    def k(l_hbm, mi_hbm, out_hbm, m_v, l_v, i_v, sem0, sem1):
        w = _wid()
        base = w * PER_W
        sems = (sem0, sem1)
        negv = jnp.full((16,), NEG, jnp.float32)
        lanes = lax.iota(jnp.int32, 16)
        lo_mask = lanes < 8

        def ldesc(c, b):
            return pltpu.make_async_copy(
                l_hbm.at[pl.ds((base + c) * 8, CH_E * 8)],
                l_v.at[b, pl.ds(0, CH_E * 8)], sems[b])

        def idesc(c, b):
            return pltpu.make_async_copy(
                mi_hbm.at[pl.ds((base + c) * 16, CH_E * 16)],
                i_v.at[b], sems[b])

        for b in range(2):
            ldesc(b * CH_E, b).start()
            idesc(b * CH_E, b).start()

        def init(i, _):
            m_v[pl.ds(i * 16, 16)] = negv
            return 0

        lax.fori_loop(0, NT * 8 // 16, init, 0)

        def pair(g, _):
            for b in range(2):
                c = g * 2 + b
                ldesc(c * CH_E, b).wait()
                idesc(c * CH_E, b).wait()

                def edge(e, _):
                    iv = i_v[b, pl.ds(e * 16, 16)]
                    lv = l_v[b, pl.ds(e * 8, 16)]
                    lsel = jnp.where(lo_mask, lv, negv)
                    mv = plsc.load_gather(m_v, [iv])
                    plsc.store_scatter(m_v, [iv], jnp.maximum(mv, lsel),
                                       mask=lo_mask)
                    return 0

                lax.fori_loop(0, CH_E, edge, 0)
                cn = jnp.minimum(c + 2, chunks - 1)
                ldesc(cn * CH_E, b).start()
                idesc(cn * CH_E, b).start()
            return 0

        lax.fori_loop(0, chunks // 2, pair, 0)
        for b in range(2):
            ldesc((chunks - 1) * CH_E, b).wait()
            idesc((chunks - 1) * CH_E, b).wait()
        pltpu.sync_copy(m_v, out_hbm.at[w])

    return k(lflat, midx)


def _sc_scatter(wps, idx2d, zeros_blk):
    chunks = PER_W // SCH

    @functools.partial(
        pl.kernel,
        out_type=jax.ShapeDtypeStruct((2 * NPASS * NT, CCOL), jnp.float32),
        mesh=_mesh(),
        scratch_types=[
            pltpu.VMEM_SHARED((NT, CCOL), jnp.float32),
            pltpu.VMEM((chunks, SCH), jnp.int32),
            pltpu.VMEM((2, SCH, CCOL), jnp.float32),
            pltpu.SemaphoreType.DMA,
            pltpu.SemaphoreType.DMA,
        ],
    )
    def k(w0_h, w1_h, w2_h, w3_h, w4_h, idx_h, z_h, out_h, acc, idx_v, v_buf,
          sem0, sem1):
        cid = lax.axis_index("c")
        sid = lax.axis_index("s")
        w = sid * 2 + cid
        row0 = w * PER_W
        sems = (sem0, sem1)
        pltpu.sync_copy(idx_h.at[pl.ds(w * chunks, chunks)], idx_v)
        for p, wp in enumerate((w0_h, w1_h, w2_h, w3_h, w4_h)):
            pltpu.sync_copy(z_h, acc.at[pl.ds(sid * 640, 640)])
            plsc.subcore_barrier()
            for b in range(2):
                pltpu.async_copy(wp.at[pl.ds(row0 + b * SCH, SCH)],
                                 v_buf.at[b], sems[b])

            def pair(g, _):
                for b in range(2):
                    c = g * 2 + b
                    pltpu.make_async_copy(wp.at[pl.ds(row0 + c * SCH, SCH)],
                                          v_buf.at[b], sems[b]).wait()
                    pltpu.sync_copy(v_buf.at[b], acc.at[idx_v.at[c]], add=True)
                    cn = jnp.minimum(c + 2, chunks - 1)
                    pltpu.async_copy(wp.at[pl.ds(row0 + cn * SCH, SCH)],
                                     v_buf.at[b], sems[b])
                return 0

            lax.fori_loop(0, chunks // 2, pair, 0)
            for b in range(2):
                pltpu.make_async_copy(
                    wp.at[pl.ds(row0 + (chunks - 1) * SCH, SCH)],
                    v_buf.at[b], sems[b]).wait()
            plsc.subcore_barrier()
            slot = cid * NPASS + p
            pltpu.sync_copy(acc.at[pl.ds(sid * 640, 640)],
                            out_h.at[pl.ds(slot * NT + sid * 640, 640)])
            plsc.subcore_barrier()

    return k(*wps, idx2d, zeros_blk)


def _sc_mgather(m2flat, midx):
    CH_E = 512
    chunks = PER_W // CH_E

    TBL = NT * 8 // 8192

    @functools.partial(
        pl.kernel,
        out_type=jax.ShapeDtypeStruct((EP * 8,), jnp.float32),
        mesh=_mesh(),
        compiler_params=pltpu.CompilerParams(needs_layout_passes=False),
        scratch_types=[
            pltpu.VMEM((NT * 8,), jnp.float32),
            pltpu.VMEM((2, CH_E * 16), jnp.int32),
            pltpu.VMEM((CH_E * 8 + 16,), jnp.float32),
            pltpu.SemaphoreType.DMA,
            pltpu.SemaphoreType.DMA,
            pltpu.SemaphoreType.DMA,
        ],
    )
    def k(m_hbm, mi_hbm, out_hbm, m_v, i_v, o_v, sem0, sem1, tsem):
        w = _wid()
        base = w * PER_W
        sems = (sem0, sem1)
        lanes = lax.iota(jnp.int32, 16)
        lo_mask = lanes < 8

        def tdesc(t):
            return pltpu.make_async_copy(m_hbm.at[pl.ds(t * 8192, 8192)],
                                         m_v.at[pl.ds(t * 8192, 8192)], tsem)

        def idesc(c, b):
            return pltpu.make_async_copy(
                mi_hbm.at[pl.ds((base + c) * 16, CH_E * 16)],
                i_v.at[b], sems[b])

        for t in range(TBL):
            tdesc(t).start()
        for b in range(2):
            idesc(b * CH_E, b).start()
        for t in range(TBL):
            tdesc(t).wait()

        def pair(g, _):
            for b in range(2):
                c = g * 2 + b
                idesc(c * CH_E, b).wait()

                def edge(e, _):
                    iv = i_v[b, pl.ds(e * 16, 16)]
                    mv = plsc.load_gather(m_v, [jnp.where(lo_mask, iv, 0)])
                    o_v[pl.ds(e * 8, 16)] = mv
                    return 0

                lax.fori_loop(0, CH_E, edge, 0)
                cn = jnp.minimum(c + 2, chunks - 1)
                idesc(cn * CH_E, b).start()
                pltpu.sync_copy(o_v.at[pl.ds(0, CH_E * 8)],
                                out_hbm.at[pl.ds((base + c * CH_E) * 8,
                                                 CH_E * 8)])
            return 0

        lax.fori_loop(0, chunks // 2, pair, 0)
        for b in range(2):
            idesc((chunks - 1) * CH_E, b).wait()

    return k(m2flat, midx)


def _dot(a, b):
    return jnp.dot(a, b, precision=HP)


def _t1_tables(s, vx, vy, vz, rot9, trans, p):
    B = 512
    grid = NT // B

    def body(s_r, vx_r, vy_r, vz_r, r9_r, tr_r, wq_r, wk_r, wv_r, wqp_r, wkp_r,
             wvp_r, vq_r, vk_r, vv_r, td_r, tsab_r):
        sb = s_r[...]
        vpl = (vx_r[...], vy_r[...], vz_r[...])
        r9 = r9_r[...]
        tr = tr_r[...]
        q = _dot(sb, wq_r[...])
        kk = _dot(sb, wk_r[...])
        vs = _dot(sb, wv_r[...])
        qp = [_dot(sb, wqp_r[j]) + _dot(vpl[j], vq_r[...]) for j in range(3)]
        kp = [_dot(sb, wkp_r[j]) + _dot(vpl[j], vk_r[...]) for j in range(3)]
        vp = [_dot(sb, wvp_r[j]) + _dot(vpl[j], vv_r[...]) for j in range(3)]

        def glob(pts, i):
            return (r9[:, 3 * i:3 * i + 1] * pts[0]
                    + r9[:, 3 * i + 1:3 * i + 2] * pts[1]
                    + r9[:, 3 * i + 2:3 * i + 3] * pts[2]
                    + tr[:, i:i + 1])

        qpg = [glob(qp, i) for i in range(3)]
        kpg = [glob(kp, i) for i in range(3)]
        vpg = [glob(vp, i) for i in range(3)]
        rid = pl.program_id(0) * B + lax.broadcasted_iota(jnp.int32, (B, 1), 0)
        valid = rid < N
        z32 = jnp.zeros((B, 32), jnp.float32)
        td = jnp.where(valid, jnp.concatenate([q] + qpg + [z32], axis=1), 0.0)
        tsab = jnp.where(
            valid,
            jnp.concatenate([kk] + kpg + [z32, vs] + vpg + [z32, z32], axis=1),
            0.0)
        td_r[...] = td
        tsab_r[...] = tsab

    full = lambda shape: pl.BlockSpec(shape, lambda i: tuple(0 for _ in shape))
    row = lambda w: pl.BlockSpec((B, w), lambda i: (i, 0))
    return pl.pallas_call(
        body,
        grid=(grid,),
        in_specs=[row(CS), row(CV), row(CV), row(CV), row(9), row(3),
                  full((CS, CS)), full((CS, CS)), full((CS, CS)),
                  full((3, CS, H * PQ)), full((3, CS, H * PQ)),
                  full((3, CS, H * PV)),
                  full((CV, H * PQ)), full((CV, H * PQ)), full((CV, H * PV))],
        out_specs=[row(256), row(640)],
        out_shape=[jax.ShapeDtypeStruct((NT, 256), jnp.float32),
                   jax.ShapeDtypeStruct((NT, 640), jnp.float32)],
    )(s, vx, vy, vz, rot9, trans, p["wq"], p["wk"], p["wv"], p["wqp3"],
      p["wkp3"], p["wvp3"], p["vq"], p["vk"], p["vv"])


def _t2_logits(td_g, tsa_g, zp, dstcol, p, selqk, seld2):
    B = 512
    grid = EP // B

    def body(td_r, tsa_r, z_r, d_r, wb_r, g_r, sq_r, sd_r, l_r, mi_r):
        td = td_r[...]
        tsa = tsa_r[...]
        z = z_r[...]
        lq = _dot(td[:, :CS] * tsa[:, :CS], sq_r[...]) * 0.25
        d2 = _dot((td[:, CS:224] - tsa[:, CS:224]) ** 2, sd_r[...])
        logits = lq + _dot(z, wb_r[...]) - 0.5 * g_r[...] * d2
        eid = pl.program_id(0) * B + lax.broadcasted_iota(jnp.int32, (B, H), 0)
        l_r[...] = jnp.where(eid < E, logits, NEG)
        mi_r[...] = d_r[...] * 8 + lax.broadcasted_iota(jnp.int32, (B, 16), 1)

    full = lambda shape: pl.BlockSpec(shape, lambda i: tuple(0 for _ in shape))
    row = lambda w: pl.BlockSpec((B, w), lambda i: (i, 0))
    return pl.pallas_call(
        body,
        grid=(grid,),
        in_specs=[row(256), row(256), row(CZ), row(1),
                  full((CZ, H)), full((1, H)), full((CS, H)), full((96, H))],
        out_specs=[row(H), row(16)],
        out_shape=[jax.ShapeDtypeStruct((EP, H), jnp.float32),
                   jax.ShapeDtypeStruct((EP, 16), jnp.int32)],
    )(td_g, tsa_g, zp, dstcol, p["wb"], p["gsp"], selqk, seld2)


def _t3_mmerge(parts):
    B = 512
    grid = NT // B

    def body(p_r, m_r):
        x = p_r[...]
        m = x[0]
        for i in range(1, NWORK):
            m = jnp.maximum(m, x[i])
        m_r[...] = m

    return pl.pallas_call(
        body,
        grid=(grid,),
        in_specs=[pl.BlockSpec((NWORK, B, 8), lambda i: (0, i, 0))],
        out_specs=pl.BlockSpec((B, 8), lambda i: (i, 0)),
        out_shape=jax.ShapeDtypeStruct((NT, 8), jnp.float32),
    )(parts)


def _t4_weights(logits, m_g, tsb_g, zp, e16, e8, e32):
    B = 512
    grid = EP // B

    def body(l_r, m_r, tsb_r, z_r, e16_r, e8_r, e32_r, w0_r, w1_r, w2_r, w3_r,
             w4_r):
        w = jnp.exp(l_r[...] - m_r[...])
        tsb = tsb_r[...][:, 256:]
        z = z_r[...]
        r16 = _dot(w, e16_r[...])
        r8 = _dot(w, e8_r[...])
        r32 = _dot(w, e32_r[...])
        ztile = jnp.concatenate([z] * H, axis=1)
        cat = jnp.concatenate(
            [r16 * tsb[:, :CS]]
            + [r8 * tsb[:, CS + 64 * j:CS + 64 * (j + 1)] for j in range(3)]
            + [r32 * ztile, w, jnp.zeros((B, 56), jnp.float32)], axis=1)
        for i, o_r in enumerate((w0_r, w1_r, w2_r, w3_r, w4_r)):
            o_r[...] = cat[:, i * CCOL:(i + 1) * CCOL]

    full = lambda shape: pl.BlockSpec(shape, lambda i: tuple(0 for _ in shape))
    row = lambda w: pl.BlockSpec((B, w), lambda i: (i, 0))
    return pl.pallas_call(
        body,
        grid=(grid,),
        in_specs=[row(H), row(H), row(640), row(CZ),
                  full((H, 128)), full((H, 64)), full((H, 256))],
        out_specs=[row(CCOL)] * NPASS,
        out_shape=[jax.ShapeDtypeStruct((EP, CCOL), jnp.float32)] * NPASS,
    )(logits, m_g, tsb_g, zp, e16, e8, e32)


def _t5_finalize(acc, s, vx, vy, vz, rot9, trans, inv, p, lnp, vlng,
                 e16, e8, e32):
    B = 400
    grid = N // B

    def body(a_r, s_r, vx_r, vy_r, vz_r, r9_r, tr_r, inv_r, wo_r, bo_r, wpv_r,
             g_r, b_r, vg_r, e16_r, e8_r, e32_r, so_r, vxo_r, vyo_r, vzo_r):
        a = a_r[...]
        acc2 = a[0] + a[1]
        flat = jnp.concatenate([acc2[j] for j in range(NPASS)], axis=1)
        den = flat[:, 576:584] + 1e-9
        d16 = _dot(den, e16_r[...])
        d8 = _dot(den, e8_r[...])
        d32 = _dot(den, e32_r[...])
        o = flat[:, :CS] / d16
        r9 = r9_r[...]
        tr = tr_r[...]
        opt = [flat[:, CS + 64 * j:CS + 64 * (j + 1)] / d8 for j in range(3)]
        optl = [sum((r9[:, 3 * j + i:3 * j + i + 1]
                     * (opt[j] - tr[:, j:j + 1])) for j in range(3))
                for i in range(3)]
        onorm = jnp.sqrt(optl[0] ** 2 + optl[1] ** 2 + optl[2] ** 2 + 1e-8)
        ozn = flat[:, 320:576] / d32
        feats = jnp.concatenate([o] + optl + [onorm, ozn], axis=1)
        su = _dot(feats, wo_r[...]) + bo_r[...]
        inv_b = inv_r[...]
        sn = s_r[...] + su * inv_b
        mu = jnp.mean(sn, axis=1, keepdims=True)
        var = jnp.mean((sn - mu) ** 2, axis=1, keepdims=True)
        so_r[...] = (sn - mu) / jnp.sqrt(var + 1e-5) * g_r[...] + b_r[...]
        vn = [v_r[...] + _dot(optl[i], wpv_r[...]) * inv_b
              for i, v_r in enumerate((vx_r, vy_r, vz_r))]
        n2 = jnp.mean(vn[0] ** 2 + vn[1] ** 2 + vn[2] ** 2, axis=1,
                      keepdims=True)
        scale = vg_r[...] / jnp.sqrt(n2 + 1e-6)
        vxo_r[...] = vn[0] * scale
        vyo_r[...] = vn[1] * scale
        vzo_r[...] = vn[2] * scale

    full = lambda shape: pl.BlockSpec(shape, lambda i: tuple(0 for _ in shape))
    row = lambda w: pl.BlockSpec((B, w), lambda i: (i, 0))
    return pl.pallas_call(
        body,
        grid=(grid,),
        in_specs=[pl.BlockSpec((2, NPASS, B, CCOL), lambda i: (0, 0, i, 0)),
                  row(CS), row(CV), row(CV), row(CV), row(9), row(3), row(1),
                  full((640, CS)), full((1, CS)), full((64, CV)),
                  full((1, CS)), full((1, CS)), full((1, CV)),
                  full((H, 128)), full((H, 64)), full((H, 256))],
        out_specs=[row(CS), row(CV), row(CV), row(CV)],
        out_shape=[jax.ShapeDtypeStruct((N, CS), jnp.float32)]
        + [jax.ShapeDtypeStruct((N, CV), jnp.float32)] * 3,
    )(acc, s, vx, vy, vz, rot9, trans, inv, p["woP"], p["bo"], p["wpv"],
      lnp["g"].reshape(1, CS), lnp["b"].reshape(1, CS), vlng, e16, e8, e32)


def _t6_post(s, vx, vy, vz, rot9, trans, inv, noise, pr):
    B = 400
    grid = N // B

    def body(s_r, vx_r, vy_r, vz_r, r9_r, tr_r, inv_r, no_r,
             lc_w1, lc_b1, lc_ws, lc_bs, lc_wg, lc_bg, lc_wm,
             lr_w1, lr_b1, lr_ws, lr_bs, lr_wg, lr_bg, lr_wm,
             l3g, l3b, v3g,
             nt_w1, nt_b1, nt_w2, nt_b2, nt_g, nt_b, nt_wg, nt_bg, nt_wm,
             w6_r, b6_r, wv2_r,
             so_r, vxo_r, vyo_r, vzo_r, r9o_r, tro_r):
        s_ = s_r[...]
        v = [vx_r[...], vy_r[...], vz_r[...]]
        inv_b = inv_r[...]
        no_b = no_r[...]

        def gate(s_, v, w1, b1, ws, bs, wg, bg, wm, act):
            nrm = jnp.sqrt(v[0] ** 2 + v[1] ** 2 + v[2] ** 2 + 1e-8)
            h = jnp.maximum(_dot(jnp.concatenate([s_, nrm], axis=1), w1[...])
                            + b1[...], 0.0)
            su = _dot(h, ws[...]) + bs[...]
            g = act(_dot(h, wg[...]) + bg[...])
            vu = [g * _dot(v[i], wm[...]) for i in range(3)]
            return su, vu

        su, vu = gate(s_, v, lc_w1, lc_b1, lc_ws, lc_bs, lc_wg, lc_bg, lc_wm,
                      jax.nn.sigmoid)
        s_ = s_ + su * inv_b
        v = [v[i] + vu[i] * inv_b for i in range(3)]
        su, vu = gate(s_, v, lr_w1, lr_b1, lr_ws, lr_bs, lr_wg, lr_bg, lr_wm,
                      jnp.tanh)
        sn = s_ + su * inv_b

        def ln(x, g, b):
            mu = jnp.mean(x, axis=1, keepdims=True)
            var = jnp.mean((x - mu) ** 2, axis=1, keepdims=True)
            return (x - mu) / jnp.sqrt(var + 1e-5) * g[...] + b[...]

        s_ = ln(sn, l3g, l3b)
        v = [v[i] + vu[i] * inv_b for i in range(3)]
        n2 = jnp.mean(v[0] ** 2 + v[1] ** 2 + v[2] ** 2, axis=1, keepdims=True)
        v = [v[i] * (v3g[...] / jnp.sqrt(n2 + 1e-6)) for i in range(3)]

        h = jnp.maximum(_dot(s_, nt_w1[...]) + nt_b1[...], 0.0)
        s2 = ln(s_ + _dot(h, nt_w2[...]) + nt_b2[...], nt_g, nt_b)
        gg = jax.nn.sigmoid(_dot(s2, nt_wg[...]) + nt_bg[...])
        v = [v[i] + gg * _dot(v[i], nt_wm[...]) for i in range(3)]

        s_f = s2 * inv_b
        v_f = [v[i] * inv_b for i in range(3)]

        sb = s_f * no_b
        vb = [v_f[i] * no_b for i in range(3)]
        a = [_dot(vb[i], wv2_r[...]) for i in range(3)]
        vc = jnp.concatenate([a[0][:, :1], a[1][:, :1], a[2][:, :1],
                              a[0][:, 1:], a[1][:, 1:], a[2][:, 1:]], axis=1)
        upd = (_dot(sb, w6_r[...]) + b6_r[...] + vc) * no_b

        qn = jnp.sqrt(1.0 + upd[:, 0:1] ** 2 + upd[:, 1:2] ** 2
                      + upd[:, 2:3] ** 2)
        qw = 1.0 / qn
        qx = upd[:, 0:1] / qn
        qy = upd[:, 1:2] / qn
        qz = upd[:, 2:3] / qn
        ru = [1 - 2 * (qy * qy + qz * qz), 2 * (qx * qy - qz * qw),
              2 * (qx * qz + qy * qw),
              2 * (qx * qy + qz * qw), 1 - 2 * (qx * qx + qz * qz),
              2 * (qy * qz - qx * qw),
              2 * (qx * qz - qy * qw), 2 * (qy * qz + qx * qw),
              1 - 2 * (qx * qx + qy * qy)]
        r9 = r9_r[...]
        newr = [sum(r9[:, 3 * i + j:3 * i + j + 1] * ru[3 * j + k]
                    for j in range(3)) for i in range(3) for k in range(3)]
        tr = tr_r[...]
        newt = [tr[:, i:i + 1]
                + sum(r9[:, 3 * i + j:3 * i + j + 1] * upd[:, 3 + j:4 + j]
                      for j in range(3)) for i in range(3)]
        so_r[...] = s_f
        vxo_r[...] = v_f[0]
        vyo_r[...] = v_f[1]
        vzo_r[...] = v_f[2]
        r9o_r[...] = jnp.concatenate(newr, axis=1)
        tro_r[...] = jnp.concatenate(newt, axis=1)

    full = lambda shape: pl.BlockSpec(shape, lambda i: tuple(0 for _ in shape))
    row = lambda w: pl.BlockSpec((B, w), lambda i: (i, 0))
    lcu, lru, nt, bb = pr["lcu"], pr["lru"], pr["nt"], pr["bb"]
    return pl.pallas_call(
        body,
        grid=(grid,),
        in_specs=[row(CS), row(CV), row(CV), row(CV), row(9), row(3), row(1),
                  row(1),
                  full((CS + CV, CS)), full((1, CS)), full((CS, CS)),
                  full((1, CS)), full((CS, CV)), full((1, CV)), full((CV, CV)),
                  full((CS + CV, CS)), full((1, CS)), full((CS, CS)),
                  full((1, CS)), full((CS, CV)), full((1, CV)), full((CV, CV)),
                  full((1, CS)), full((1, CS)), full((1, CV)),
                  full((CS, 2 * CS)), full((1, 2 * CS)), full((2 * CS, CS)),
                  full((1, CS)), full((1, CS)), full((1, CS)),
                  full((CS, CV)), full((1, CV)), full((CV, CV)),
                  full((CS, 6)), full((1, 6)), full((CV, 2))],
        out_specs=[row(CS), row(CV), row(CV), row(CV), row(9), row(3)],
        out_shape=[jax.ShapeDtypeStruct((N, CS), jnp.float32),
                   jax.ShapeDtypeStruct((N, CV), jnp.float32),
                   jax.ShapeDtypeStruct((N, CV), jnp.float32),
                   jax.ShapeDtypeStruct((N, CV), jnp.float32),
                   jax.ShapeDtypeStruct((N, 9), jnp.float32),
                   jax.ShapeDtypeStruct((N, 3), jnp.float32)],
    )(s, vx, vy, vz, rot9, trans, inv, noise,
      lcu["w1"], lcu["b1"], lcu["ws"], lcu["bs"], lcu["wg"], lcu["bg"],
      lcu["wm"],
      lru["w1"], lru["b1"], lru["ws"], lru["bs"], lru["wg"], lru["bg"],
      lru["wm"],
      pr["ln_s3g"], pr["ln_s3b"], pr["ln_v3"],
      nt["w1"], nt["b1"], nt["w2"], nt["b2"], nt["g"], nt["b"], nt["wg"],
      nt["bg"], nt["wm"],
      bb["w6"], bb["b6"], bb["wv2"])


def _t7_edge_tr(sg, piece_src, piece_dst, z, p):
    B = 512
    grid = (E + B - 1) // B
    off_s = piece_src * (EP // B)
    off_d = piece_dst * (EP // B)

    def body(ss_r, sd_r, z_r, w1_r, b1_r, w2_r, b2_r, g_r, b_r, zo_r):
        z_ = z_r[...]
        hcat = jnp.concatenate([ss_r[...], sd_r[...], z_], axis=1)
        h = jnp.maximum(_dot(hcat, w1_r[...]) + b1_r[...], 0.0)
        zn = z_ + _dot(h, w2_r[...]) + b2_r[...]
        mu = jnp.mean(zn, axis=1, keepdims=True)
        var = jnp.mean((zn - mu) ** 2, axis=1, keepdims=True)
        zo_r[...] = (zn - mu) / jnp.sqrt(var + 1e-5) * g_r[...] + b_r[...]

    full = lambda shape: pl.BlockSpec(shape, lambda i: tuple(0 for _ in shape))
    return pl.pallas_call(
        body,
        grid=(grid,),
        in_specs=[pl.BlockSpec((B, CS), lambda i: (i + off_s, 0)),
                  pl.BlockSpec((B, CS), lambda i: (i + off_d, 0)),
                  pl.BlockSpec((B, CZ), lambda i: (i, 0)),
                  full((2 * CS + CZ, 2 * CZ)), full((1, 2 * CZ)),
                  full((2 * CZ, CZ)), full((1, CZ)),
                  full((1, CZ)), full((1, CZ))],
        out_specs=pl.BlockSpec((B, CZ), lambda i: (i, 0)),
        out_shape=jax.ShapeDtypeStruct((E, CZ), jnp.float32),
    )(sg, sg, z, p["w1"], p["b1"], p["w2"], p["b2"], p["g"], p["b"])


def _prep_psa(p):
    wo = p["wo"]
    o_part = wo[:CS]
    optl_part = wo[CS:CS + 192].reshape(64, 3, CS).transpose(1, 0, 2).reshape(192, CS)
    rest = wo[CS + 192:]
    return {
        "wq": p["wq"], "wk": p["wk"], "wv": p["wv"],
        "wqp3": p["wqp"].reshape(CS, H * PQ, 3).transpose(2, 0, 1),
        "wkp3": p["wkp"].reshape(CS, H * PQ, 3).transpose(2, 0, 1),
        "wvp3": p["wvp"].reshape(CS, H * PV, 3).transpose(2, 0, 1),
        "vq": p["vq"], "vk": p["vk"], "vv": p["vv"],
        "wb": p["wb"],
        "gsp": jax.nn.softplus(p["gamma"]).reshape(1, H),
        "woP": jnp.concatenate([o_part, optl_part, rest], axis=0),
        "bo": p["bo"].reshape(1, CS),
        "wpv": p["wpv"],
    }


def _pad_idx(idx, pad_val, rows):
    return jnp.pad(idx, (0, rows - idx.shape[0]), constant_values=pad_val)


def _attention(s, vx, vy, vz, rot9, trans, inv, z, ei, pp, lnp, vlng, consts):
    e16, e8, e32, selqk, seld2, zeros_blk = consts
    src = _pad_idx(ei[0], N, EP)
    dst = _pad_idx(ei[1], N, EP)
    dst2d_s = dst.reshape(EP // SCH, SCH)
    zp = jnp.pad(z, ((0, EP - E), (0, 0)))

    td, tsab = _t1_tables(s, vx, vy, vz, rot9, trans, pp)
    td_g = _sc_gather(td, dst, 256, EP)
    tsab_g = _sc_gather(tsab, src, 640, EP)
    logits, midx = _t2_logits(td_g, tsab_g, zp, dst.reshape(EP, 1), pp,
                              selqk, seld2)
    midx_f = midx.reshape(EP * 16)
    parts = _sc_segmax(logits.reshape(EP * H), midx_f)
    m2 = _t3_mmerge(parts.reshape(NWORK, NT, 8))
    m_g = _sc_mgather(m2.reshape(NT * 8), midx_f).reshape(EP, 8)
    wps = _t4_weights(logits, m_g, tsab_g, zp, e16, e8, e32)
    acc = _sc_scatter(wps, dst2d_s, zeros_blk)
    return _t5_finalize(acc.reshape(2, NPASS, NT, CCOL), s, vx, vy, vz, rot9,
                        trans, inv, pp, lnp, vlng, e16, e8, e32)


def kernel(node_features, rigids_rot, rigids_trans, edge_features, edge_index,
           seq_edge_features, seq_edge_index, x_mask, noising_mask,
           node_vectors, params):
    f32 = jnp.float32
    s0 = node_features
    rot9 = rigids_rot.reshape(N, 9)
    trans = rigids_trans
    inv = (~x_mask).astype(f32).reshape(N, 1)
    noise = noising_mask.reshape(N, 1)
    v_pl = jnp.transpose(node_vectors, (2, 0, 1))
    vx, vy, vz = v_pl[0], v_pl[1], v_pl[2]

    hh = jnp.arange(H)
    e16 = (jnp.arange(128)[None, :] // 16 == hh[:, None]).astype(f32)
    e8 = (jnp.arange(64)[None, :] // 8 == hh[:, None]).astype(f32)
    e32 = (jnp.arange(256)[None, :] // 32 == hh[:, None]).astype(f32)
    selqk = e16.T
    seld2 = ((jnp.arange(96)[:, None] % 32) // 4 == hh[None, :]).astype(f32)
    zeros_blk = jnp.zeros((640, CCOL), f32)
    consts = (e16, e8, e32, selqk, seld2, zeros_blk)

    pA = _prep_psa(params["attn_seq"])
    pB = _prep_psa(params["attn_spatial"])

    s1, vx1, vy1, vz1 = _attention(
        s0, vx, vy, vz, rot9, trans, inv, seq_edge_features, seq_edge_index,
        pA, params["ln_s1"], params["ln_v1"].reshape(1, CV), consts)
    s2, vx2, vy2, vz2 = _attention(
        s1, vx1, vy1, vz1, rot9, trans, inv, edge_features, edge_index,
        pB, params["ln_s2"], params["ln_v2"].reshape(1, CV), consts)

    pr = {
        "lcu": {k: (v.reshape(1, -1) if v.ndim == 1 else v)
                for k, v in params["lcu"].items()},
        "lru": {k: (v.reshape(1, -1) if v.ndim == 1 else v)
                for k, v in params["lru"].items()},
        "nt": {k: (v.reshape(1, -1) if v.ndim == 1 else v)
               for k, v in params["nt"].items()},
        "bb": {"w6": params["bb"]["w6"], "b6": params["bb"]["b6"].reshape(1, 6),
               "wv2": params["bb"]["wv2"]},
        "ln_s3g": params["ln_s3"]["g"].reshape(1, CS),
        "ln_s3b": params["ln_s3"]["b"].reshape(1, CS),
        "ln_v3": params["ln_v3"].reshape(1, CV),
    }
    s3, vfx, vfy, vfz, r9n, trn = _t6_post(s2, vx2, vy2, vz2, rot9, trans,
                                           inv, noise, pr)

    srcB = _pad_idx(edge_index[0], 0, EP)
    dstB = _pad_idx(edge_index[1], 0, EP)
    srcA = _pad_idx(seq_edge_index[0], 0, EP)
    dstA = _pad_idx(seq_edge_index[1], 0, EP)
    idx_et = jnp.concatenate([srcB, dstB, srcA, dstA])
    sg = _sc_gather(s3, idx_et, CS, 4 * EP)

    et = {k: (v.reshape(1, -1) if v.ndim == 1 else v)
          for k, v in params["et"].items()}
    set_ = {k: (v.reshape(1, -1) if v.ndim == 1 else v)
            for k, v in params["set"].items()}
    z_out = _t7_edge_tr(sg, 0, 1, edge_features, et)
    zs_out = _t7_edge_tr(sg, 2, 3, seq_edge_features, set_)

    v_final = jnp.stack([vfx, vfy, vfz], axis=-1)
    return (s3, r9n.reshape(N, 3, 3), trn, z_out, zs_out, v_final)
